# baseline (speedup 1.0000x reference)
import functools

import jax
import jax.numpy as jnp
from jax import lax
from jax.experimental import pallas as pl
from jax.experimental.pallas import tpu as pltpu
from jax.experimental.pallas import tpu_sc as plsc

_pc = pl.pallas_call

N = 10000
E = 320000
D = 128
D2 = 256
DEF = 48
LAYERS = 3
NW = 32
EPW = E // NW
K = 80
NCH = EPW // K
RPT = N // 16

BE = 4000
BN_ = 2000


def _rbf_body(r_ref, ef_ref):
    r = r_ref[...]
    bl = jnp.sqrt(jnp.sum(r * r, axis=1, keepdims=True))
    lane_i = lax.broadcasted_iota(jnp.int32, (BE, DEF), 1)
    c = (lane_i.astype(jnp.float32) - 1.0) * (8.0 / 39.0)
    gamma = (39.0 / 8.0) ** 2
    val = jnp.exp(-gamma * (bl - c) ** 2)
    ef = jnp.where(lane_i == 0, 1.0, jnp.where(lane_i <= 40, val, 0.0))
    ef_ref[...] = ef


def _rbf(r):
    return _pc(
        _rbf_body,
        grid=(E // BE,),
        in_specs=[pl.BlockSpec((BE, 3), lambda i: (i, 0))],
        out_specs=pl.BlockSpec((BE, DEF), lambda i: (i, 0)),
        out_shape=jax.ShapeDtypeStruct((E, DEF), jnp.float32),
    )(r)


def _embed_body(at_ref, af_ref, ew_ref, eb_ref, ws_ref, bs_ref, wd_ref, bd_ref,
                x_ref, hs_ref, hd_ref):
    t = at_ref[...]
    oh = (lax.broadcasted_iota(jnp.int32, (BN_, 128), 1) == t).astype(jnp.float32)
    v = jnp.dot(oh, af_ref[...], preferred_element_type=jnp.float32)
    x = jnp.dot(v, ew_ref[...], preferred_element_type=jnp.float32) + eb_ref[...]
    x_ref[...] = x
    hs_ref[...] = (jnp.dot(x, ws_ref[...], preferred_element_type=jnp.float32)
                   + bs_ref[...]).astype(jnp.bfloat16)
    hd_ref[...] = (jnp.dot(x, wd_ref[...], preferred_element_type=jnp.float32)
                   + bd_ref[...]).astype(jnp.bfloat16)


def _embed(at, af_pad, ew, eb, ws, bs, wd, bd):
    da = af_pad.shape[1]
    return _pc(
        _embed_body,
        grid=(N // BN_,),
        in_specs=[
            pl.BlockSpec((BN_, 1), lambda i: (i, 0)),
            pl.BlockSpec((128, da), lambda i: (0, 0)),
            pl.BlockSpec((da, D), lambda i: (0, 0)),
            pl.BlockSpec((1, D), lambda i: (0, 0)),
            pl.BlockSpec((D, D2), lambda i: (0, 0)),
            pl.BlockSpec((1, D2), lambda i: (0, 0)),
            pl.BlockSpec((D, D2), lambda i: (0, 0)),
            pl.BlockSpec((1, D2), lambda i: (0, 0)),
        ],
        out_specs=[
            pl.BlockSpec((BN_, D), lambda i: (i, 0)),
            pl.BlockSpec((BN_, D2), lambda i: (i, 0)),
            pl.BlockSpec((BN_, D2), lambda i: (i, 0)),
        ],
        out_shape=[
            jax.ShapeDtypeStruct((N, D), jnp.float32),
            jax.ShapeDtypeStruct((N, D2), jnp.bfloat16),
            jax.ShapeDtypeStruct((N, D2), jnp.bfloat16),
        ],
    )(at, af_pad, ew, eb, ws, bs, wd, bd)


def _sc_gather(hs_i, hd_i, src, dst):
    DW = D2 // 2
    mesh = plsc.VectorSubcoreMesh(core_axis_name="c", subcore_axis_name="s")

    @functools.partial(
        pl.kernel,
        out_type=[
            jax.ShapeDtypeStruct((E, DW), jnp.int32),
            jax.ShapeDtypeStruct((E, DW), jnp.int32),
        ],
        mesh=mesh,
        scratch_types=[
            pltpu.VMEM((K,), jnp.int32),
            pltpu.VMEM((K,), jnp.int32),
            pltpu.VMEM((K, DW), jnp.int32),
            pltpu.VMEM((K, DW), jnp.int32),
            pltpu.SemaphoreType.DMA,
            pltpu.SemaphoreType.DMA,
        ],
    )
    def k(hs_hbm, hd_hbm, src_hbm, dst_hbm, ga_hbm, gb_hbm, isrc, idst,
          abuf, bbuf, sema, semb):
        cid = lax.axis_index("c")
        sid = lax.axis_index("s")
        base = (cid * 16 + sid) * EPW

        def chunk(j, carry):
            e0 = base + j * K
            pltpu.sync_copy(src_hbm.at[pl.ds(e0, K)], isrc)
            pltpu.sync_copy(dst_hbm.at[pl.ds(e0, K)], idst)
            ca = pltpu.async_copy(hs_hbm.at[isrc], abuf, sema)
            cb = pltpu.async_copy(hd_hbm.at[idst], bbuf, semb)
            ca.wait()
            pltpu.sync_copy(abuf, ga_hbm.at[pl.ds(e0, K)])
            cb.wait()
            pltpu.sync_copy(bbuf, gb_hbm.at[pl.ds(e0, K)])
            return carry

        lax.fori_loop(0, NCH, chunk, 0)

    return k(hs_i, hd_i, src, dst)


def _sc_scatter(g, dst):
    mesh = plsc.VectorSubcoreMesh(core_axis_name="c", subcore_axis_name="s")

    @functools.partial(
        pl.kernel,
        out_type=jax.ShapeDtypeStruct((2, 16, RPT, D), jnp.float32),
        mesh=mesh,
        scratch_types=[
            pltpu.VMEM((K,), jnp.int32),
            pltpu.VMEM((K, D), jnp.float32),
            pltpu.VMEM((RPT // 5, D), jnp.float32),
            pltpu.VMEM_SHARED((N, D), jnp.float32),
        ],
    )
    def k(g_hbm, dst_hbm, out_hbm, idx, gbuf, zbuf, acc):
        cid = lax.axis_index("c")
        sid = lax.axis_index("s")
        wid = cid * 16 + sid

        zr = RPT // 5

        def zrow(i, c2):
            for q in range(D // 16):
                zbuf[i, pl.ds(q * 16, 16)] = jnp.zeros((16,), jnp.float32)
            return c2

        lax.fori_loop(0, zr, zrow, 0)

        def zcp(i, c2):
            pltpu.sync_copy(zbuf, acc.at[pl.ds(sid * RPT + i * zr, zr)])
            return c2

        lax.fori_loop(0, 5, zcp, 0)
        plsc.subcore_barrier()

        def chunk(j, carry):
            e0 = wid * EPW + j * K
            pltpu.sync_copy(dst_hbm.at[pl.ds(e0, K)], idx)
            pltpu.sync_copy(g_hbm.at[pl.ds(e0, K)], gbuf)
            pltpu.sync_copy(gbuf, acc.at[idx], add=True)
            return carry

        lax.fori_loop(0, NCH, chunk, 0)
        plsc.subcore_barrier()
        pltpu.sync_copy(acc.at[pl.ds(sid * RPT, RPT)], out_hbm.at[cid, sid])

    return k(g, dst).reshape(2, N, D)


def _stats_body(sa_ref, sb_ref, ef_ref, w_ref, be_ref, sum_ref, ssq_ref):
    i = pl.program_id(0)
    c = jnp.dot(ef_ref[...], w_ref[...], preferred_element_type=jnp.float32)
    m = (sa_ref[...].astype(jnp.float32) + sb_ref[...].astype(jnp.float32)
         + c + be_ref[...])
    s0 = jnp.pad(jnp.sum(m, axis=0, keepdims=True), ((0, 7), (0, 0)))
    s1 = jnp.pad(jnp.sum(m * m, axis=0, keepdims=True), ((0, 7), (0, 0)))

    @pl.when(i == 0)
    def _():
        sum_ref[...] = jnp.zeros_like(sum_ref)
        ssq_ref[...] = jnp.zeros_like(ssq_ref)

    sum_ref[...] += s0
    ssq_ref[...] += s1


def _stats(sa, sb, ef, w_aug, be):
    return _pc(
        _stats_body,
        grid=(E // BE,),
        in_specs=[
            pl.BlockSpec((BE, D2), lambda i: (i, 0)),
            pl.BlockSpec((BE, D2), lambda i: (i, 0)),
            pl.BlockSpec((BE, DEF), lambda i: (i, 0)),
            pl.BlockSpec((DEF, D2), lambda i: (0, 0)),
            pl.BlockSpec((1, D2), lambda i: (0, 0)),
        ],
        out_specs=[
            pl.BlockSpec((8, D2), lambda i: (0, 0)),
            pl.BlockSpec((8, D2), lambda i: (0, 0)),
        ],
        out_shape=[
            jax.ShapeDtypeStruct((8, D2), jnp.float32),
            jax.ShapeDtypeStruct((8, D2), jnp.float32),
        ],
    )(sa, sb, ef, w_aug, be)


def _gate_body(sa_ref, sb_ref, ef_ref, w_ref, be_ref, sum_ref, ssq_ref,
               g_ref, b_ref, out_ref):
    c = jnp.dot(ef_ref[...], w_ref[...], preferred_element_type=jnp.float32)
    m = (sa_ref[...].astype(jnp.float32) + sb_ref[...].astype(jnp.float32)
         + c + be_ref[...])
    mu = sum_ref[0:1, :] * (1.0 / E)
    var = ssq_ref[0:1, :] * (1.0 / E) - mu * mu
    rstd = lax.rsqrt(var + 1e-5)
    scale = g_ref[...] * rstd
    shift = b_ref[...] - mu * scale
    mh = m * scale + shift
    hf = mh[:, :D]
    hs = mh[:, D:]
    out_ref[...] = jax.nn.sigmoid(hf) * jax.nn.softplus(hs)


def _gate(sa, sb, ef, w_aug, be, sums, ssqs, bg, bb):
    return _pc(
        _gate_body,
        grid=(E // BE,),
        in_specs=[
            pl.BlockSpec((BE, D2), lambda i: (i, 0)),
            pl.BlockSpec((BE, D2), lambda i: (i, 0)),
            pl.BlockSpec((BE, DEF), lambda i: (i, 0)),
            pl.BlockSpec((DEF, D2), lambda i: (0, 0)),
            pl.BlockSpec((1, D2), lambda i: (0, 0)),
            pl.BlockSpec((8, D2), lambda i: (0, 0)),
            pl.BlockSpec((8, D2), lambda i: (0, 0)),
            pl.BlockSpec((1, D2), lambda i: (0, 0)),
            pl.BlockSpec((1, D2), lambda i: (0, 0)),
        ],
        out_specs=pl.BlockSpec((BE, D), lambda i: (i, 0)),
        out_shape=jax.ShapeDtypeStruct((E, D), jnp.float32),
    )(sa, sb, ef, w_aug, be, sums, ssqs, bg, bb)


def _node_core(x_ref, hp_ref, g_ref, b_ref):
    h = hp_ref[0] + hp_ref[1]
    mu = jnp.mean(h, axis=0, keepdims=True)
    d = h - mu
    var = jnp.mean(d * d, axis=0, keepdims=True)
    hn = d * lax.rsqrt(var + 1e-5) * g_ref[...] + b_ref[...]
    return jax.nn.softplus(x_ref[...] + hn)


def _node_body(x_ref, hp_ref, g_ref, b_ref, ws_ref, bs_ref, wd_ref, bd_ref,
               x2_ref, hs_ref, hd_ref):
    x2 = _node_core(x_ref, hp_ref, g_ref, b_ref)
    x2_ref[...] = x2
    hs_ref[...] = (jnp.dot(x2, ws_ref[...], preferred_element_type=jnp.float32)
                   + bs_ref[...]).astype(jnp.bfloat16)
    hd_ref[...] = (jnp.dot(x2, wd_ref[...], preferred_element_type=jnp.float32)
                   + bd_ref[...]).astype(jnp.bfloat16)


def _node(x, hp, bg, bb, ws, bs, wd, bd):
    return _pc(
        _node_body,
        in_specs=[
            pl.BlockSpec((N, D), lambda: (0, 0)),
            pl.BlockSpec((2, N, D), lambda: (0, 0, 0)),
            pl.BlockSpec((1, D), lambda: (0, 0)),
            pl.BlockSpec((1, D), lambda: (0, 0)),
            pl.BlockSpec((D, D2), lambda: (0, 0)),
            pl.BlockSpec((1, D2), lambda: (0, 0)),
            pl.BlockSpec((D, D2), lambda: (0, 0)),
            pl.BlockSpec((1, D2), lambda: (0, 0)),
        ],
        out_specs=[
            pl.BlockSpec((N, D), lambda: (0, 0)),
            pl.BlockSpec((N, D2), lambda: (0, 0)),
            pl.BlockSpec((N, D2), lambda: (0, 0)),
        ],
        out_shape=[
            jax.ShapeDtypeStruct((N, D), jnp.float32),
            jax.ShapeDtypeStruct((N, D2), jnp.bfloat16),
            jax.ShapeDtypeStruct((N, D2), jnp.bfloat16),
        ],
    )(x, hp, bg, bb, ws, bs, wd, bd)


def _final_body(x_ref, hp_ref, g_ref, b_ref, fw_ref, fb_ref, out_ref):
    x2 = _node_core(x_ref, hp_ref, g_ref, b_ref)
    feat = jnp.mean(x2, axis=0, keepdims=True)
    out_ref[...] = jnp.dot(feat, fw_ref[...], preferred_element_type=jnp.float32) + fb_ref[...]


def _final(x, hp, bg, bb, fw, fb):
    return _pc(
        _final_body,
        in_specs=[
            pl.BlockSpec((N, D), lambda: (0, 0)),
            pl.BlockSpec((2, N, D), lambda: (0, 0, 0)),
            pl.BlockSpec((1, D), lambda: (0, 0)),
            pl.BlockSpec((1, D), lambda: (0, 0)),
            pl.BlockSpec((D, 1), lambda: (0, 0)),
            pl.BlockSpec((1, 1), lambda: (0, 0)),
        ],
        out_specs=pl.BlockSpec((1, 1), lambda: (0, 0)),
        out_shape=jax.ShapeDtypeStruct((1, 1), jnp.float32),
    )(x, hp, bg, bb, fw, fb)


def _b2i(a):
    return lax.bitcast_convert_type(a.reshape(a.shape[0], -1, 2), jnp.int32)


def _i2b(a):
    return lax.bitcast_convert_type(a, jnp.bfloat16).reshape(a.shape[0], -1)


def kernel(atom_types, edge_index, r, af_table, emb_W, emb_b, W_src, b_src,
           W_dst, b_dst, W_edge, b_edge, bn_m_g, bn_m_b, bn_g, bn_b,
           fc_out_W, fc_out_b):
    src = edge_index[0].astype(jnp.int32)
    dst = edge_index[1].astype(jnp.int32)

    ef = _rbf(r)
    af_pad = jnp.pad(af_table, ((0, 128 - af_table.shape[0]), (0, 0)))
    x, hs, hd = _embed(
        atom_types.astype(jnp.int32).reshape(N, 1), af_pad, emb_W,
        emb_b.reshape(1, D), W_src[0], b_src[0].reshape(1, D2),
        W_dst[0], b_dst[0].reshape(1, D2))

    out = None
    for i in range(LAYERS):
        ga, gb = _sc_gather(_b2i(hs), _b2i(hd), src, dst)
        sa, sb = _i2b(ga), _i2b(gb)
        w_aug = jnp.pad(W_edge[i], ((1, DEF - 1 - W_edge.shape[1]), (0, 0)))
        be = b_edge[i].reshape(1, D2)
        sums, ssqs = _stats(sa, sb, ef, w_aug, be)
        g = _gate(sa, sb, ef, w_aug, be, sums, ssqs,
                  bn_m_g[i].reshape(1, D2), bn_m_b[i].reshape(1, D2))
        hp = _sc_scatter(g, dst)
        bg = bn_g[i].reshape(1, D)
        bb = bn_b[i].reshape(1, D)
        if i < LAYERS - 1:
            x, hs, hd = _node(x, hp, bg, bb, W_src[i + 1],
                              b_src[i + 1].reshape(1, D2), W_dst[i + 1],
                              b_dst[i + 1].reshape(1, D2))
        else:
            out = _final(x, hp, bg, bb, fc_out_W, fc_out_b.reshape(1, 1))
    return out


# speedup vs baseline: 3.4234x; 3.4234x over previous
import functools

import jax
import jax.numpy as jnp
from jax import lax
from jax.experimental import pallas as pl
from jax.experimental.pallas import tpu as pltpu
from jax.experimental.pallas import tpu_sc as plsc

_pc = pl.pallas_call

N = 10000
E = 320000
D = 128
D2 = 256
DEF = 48
LAYERS = 3
NW = 32
EPW = E // NW
K = 80
NCH = EPW // K
RPT = N // 16

BE = 4000
BN_ = 2000


def _rbf_body(r_ref, ef_ref):
    r = r_ref[...]
    bl = jnp.sqrt(jnp.sum(r * r, axis=1, keepdims=True))
    lane_i = lax.broadcasted_iota(jnp.int32, (BE, DEF), 1)
    c = (lane_i.astype(jnp.float32) - 1.0) * (8.0 / 39.0)
    gamma = (39.0 / 8.0) ** 2
    val = jnp.exp(-gamma * (bl - c) ** 2)
    ef = jnp.where(lane_i == 0, 1.0, jnp.where(lane_i <= 40, val, 0.0))
    ef_ref[...] = ef


def _rbf(r):
    return _pc(
        _rbf_body,
        grid=(E // BE,),
        in_specs=[pl.BlockSpec((BE, 3), lambda i: (i, 0))],
        out_specs=pl.BlockSpec((BE, DEF), lambda i: (i, 0)),
        out_shape=jax.ShapeDtypeStruct((E, DEF), jnp.float32),
    )(r)


def _pack_words(h):
    u_lo = lax.bitcast_convert_type(h[:, :D], jnp.int32)
    u_hi = lax.bitcast_convert_type(h[:, D:], jnp.int32)
    r_lo = u_lo + 0x7FFF + jnp.bitwise_and(jnp.right_shift(u_lo, 16), 1)
    r_lo = jnp.bitwise_and(jnp.right_shift(r_lo, 16), 0xFFFF)
    r_hi = u_hi + 0x7FFF + jnp.bitwise_and(jnp.right_shift(u_hi, 16), 1)
    r_hi = jnp.bitwise_and(r_hi, -65536)
    return jnp.bitwise_or(r_lo, r_hi)


def _unpack_words(w):
    lo = lax.bitcast_convert_type(jnp.left_shift(w, 16), jnp.float32)
    hi = lax.bitcast_convert_type(jnp.bitwise_and(w, -65536), jnp.float32)
    return lo, hi


def _embed_body(at_ref, af_ref, ew_ref, eb_ref, ws_ref, bs_ref, wd_ref, bd_ref,
                x_ref, hs_ref, hd_ref):
    t = at_ref[...]
    oh = (lax.broadcasted_iota(jnp.int32, (BN_, 128), 1) == t).astype(jnp.float32)
    v = jnp.dot(oh, af_ref[...], preferred_element_type=jnp.float32)
    x = jnp.dot(v, ew_ref[...], preferred_element_type=jnp.float32) + eb_ref[...]
    x_ref[...] = x
    hs_ref[...] = _pack_words(
        jnp.dot(x, ws_ref[...], preferred_element_type=jnp.float32) + bs_ref[...])
    hd_ref[...] = _pack_words(
        jnp.dot(x, wd_ref[...], preferred_element_type=jnp.float32) + bd_ref[...])


def _embed(at, af_pad, ew, eb, ws, bs, wd, bd):
    da = af_pad.shape[1]
    return _pc(
        _embed_body,
        grid=(N // BN_,),
        in_specs=[
            pl.BlockSpec((BN_, 1), lambda i: (i, 0)),
            pl.BlockSpec((128, da), lambda i: (0, 0)),
            pl.BlockSpec((da, D), lambda i: (0, 0)),
            pl.BlockSpec((1, D), lambda i: (0, 0)),
            pl.BlockSpec((D, D2), lambda i: (0, 0)),
            pl.BlockSpec((1, D2), lambda i: (0, 0)),
            pl.BlockSpec((D, D2), lambda i: (0, 0)),
            pl.BlockSpec((1, D2), lambda i: (0, 0)),
        ],
        out_specs=[
            pl.BlockSpec((BN_, D), lambda i: (i, 0)),
            pl.BlockSpec((BN_, D), lambda i: (i, 0)),
            pl.BlockSpec((BN_, D), lambda i: (i, 0)),
        ],
        out_shape=[
            jax.ShapeDtypeStruct((N, D), jnp.float32),
            jax.ShapeDtypeStruct((N, D), jnp.int32),
            jax.ShapeDtypeStruct((N, D), jnp.int32),
        ],
    )(at, af_pad, ew, eb, ws, bs, wd, bd)


def _sc_gather(hs_i, hd_i, src, dst):
    DW = D2 // 2
    mesh = plsc.VectorSubcoreMesh(core_axis_name="c", subcore_axis_name="s")

    @functools.partial(
        pl.kernel,
        out_type=[
            jax.ShapeDtypeStruct((E, DW), jnp.int32),
            jax.ShapeDtypeStruct((E, DW), jnp.int32),
        ],
        mesh=mesh,
        scratch_types=[
            pltpu.VMEM((K,), jnp.int32),
            pltpu.VMEM((K,), jnp.int32),
            pltpu.VMEM((K, DW), jnp.int32),
            pltpu.VMEM((K, DW), jnp.int32),
            pltpu.SemaphoreType.DMA,
            pltpu.SemaphoreType.DMA,
        ],
    )
    def k(hs_hbm, hd_hbm, src_hbm, dst_hbm, ga_hbm, gb_hbm, isrc, idst,
          abuf, bbuf, sema, semb):
        cid = lax.axis_index("c")
        sid = lax.axis_index("s")
        base = (cid * 16 + sid) * EPW

        def chunk(j, carry):
            e0 = base + j * K
            pltpu.sync_copy(src_hbm.at[pl.ds(e0, K)], isrc)
            pltpu.sync_copy(dst_hbm.at[pl.ds(e0, K)], idst)
            ca = pltpu.async_copy(hs_hbm.at[isrc], abuf, sema)
            cb = pltpu.async_copy(hd_hbm.at[idst], bbuf, semb)
            ca.wait()
            pltpu.sync_copy(abuf, ga_hbm.at[pl.ds(e0, K)])
            cb.wait()
            pltpu.sync_copy(bbuf, gb_hbm.at[pl.ds(e0, K)])
            return carry

        lax.fori_loop(0, NCH, chunk, 0)

    return k(hs_i, hd_i, src, dst)


def _sc_scatter(g, dst):
    mesh = plsc.VectorSubcoreMesh(core_axis_name="c", subcore_axis_name="s")

    @functools.partial(
        pl.kernel,
        out_type=jax.ShapeDtypeStruct((2, 16, RPT, D), jnp.float32),
        mesh=mesh,
        scratch_types=[
            pltpu.VMEM((K,), jnp.int32),
            pltpu.VMEM((K, D), jnp.float32),
            pltpu.VMEM((RPT // 5, D), jnp.float32),
            pltpu.VMEM_SHARED((N, D), jnp.float32),
        ],
    )
    def k(g_hbm, dst_hbm, out_hbm, idx, gbuf, zbuf, acc):
        cid = lax.axis_index("c")
        sid = lax.axis_index("s")
        wid = cid * 16 + sid

        zr = RPT // 5

        def zrow(i, c2):
            for q in range(D // 16):
                zbuf[i, pl.ds(q * 16, 16)] = jnp.zeros((16,), jnp.float32)
            return c2

        lax.fori_loop(0, zr, zrow, 0)

        def zcp(i, c2):
            pltpu.sync_copy(zbuf, acc.at[pl.ds(sid * RPT + i * zr, zr)])
            return c2

        lax.fori_loop(0, 5, zcp, 0)
        plsc.subcore_barrier()

        def chunk(j, carry):
            e0 = wid * EPW + j * K
            pltpu.sync_copy(dst_hbm.at[pl.ds(e0, K)], idx)
            pltpu.sync_copy(g_hbm.at[pl.ds(e0, K)], gbuf)
            pltpu.sync_copy(gbuf, acc.at[idx], add=True)
            return carry

        lax.fori_loop(0, NCH, chunk, 0)
        plsc.subcore_barrier()
        pltpu.sync_copy(acc.at[pl.ds(sid * RPT, RPT)], out_hbm.at[cid, sid])

    return k(g, dst).reshape(2, N, D)


def _stats_body(sa_ref, sb_ref, ef_ref, w_ref, be_ref, sum_ref, ssq_ref):
    i = pl.program_id(0)
    c = jnp.dot(ef_ref[...], w_ref[...], preferred_element_type=jnp.float32)
    alo, ahi = _unpack_words(sa_ref[...])
    blo, bhi = _unpack_words(sb_ref[...])
    m_lo = alo + blo + c[:, :D] + be_ref[:, :D]
    m_hi = ahi + bhi + c[:, D:] + be_ref[:, D:]
    m = jnp.concatenate([m_lo, m_hi], axis=1)
    s0 = jnp.pad(jnp.sum(m, axis=0, keepdims=True), ((0, 7), (0, 0)))
    s1 = jnp.pad(jnp.sum(m * m, axis=0, keepdims=True), ((0, 7), (0, 0)))

    @pl.when(i == 0)
    def _():
        sum_ref[...] = jnp.zeros_like(sum_ref)
        ssq_ref[...] = jnp.zeros_like(ssq_ref)

    sum_ref[...] += s0
    ssq_ref[...] += s1


def _stats(sa, sb, ef, w_aug, be):
    return _pc(
        _stats_body,
        grid=(E // BE,),
        in_specs=[
            pl.BlockSpec((BE, D), lambda i: (i, 0)),
            pl.BlockSpec((BE, D), lambda i: (i, 0)),
            pl.BlockSpec((BE, DEF), lambda i: (i, 0)),
            pl.BlockSpec((DEF, D2), lambda i: (0, 0)),
            pl.BlockSpec((1, D2), lambda i: (0, 0)),
        ],
        out_specs=[
            pl.BlockSpec((8, D2), lambda i: (0, 0)),
            pl.BlockSpec((8, D2), lambda i: (0, 0)),
        ],
        out_shape=[
            jax.ShapeDtypeStruct((8, D2), jnp.float32),
            jax.ShapeDtypeStruct((8, D2), jnp.float32),
        ],
    )(sa, sb, ef, w_aug, be)


def _gate_body(sa_ref, sb_ref, ef_ref, w_ref, be_ref, sum_ref, ssq_ref,
               g_ref, b_ref, out_ref):
    c = jnp.dot(ef_ref[...], w_ref[...], preferred_element_type=jnp.float32)
    alo, ahi = _unpack_words(sa_ref[...])
    blo, bhi = _unpack_words(sb_ref[...])
    m_lo = alo + blo + c[:, :D] + be_ref[:, :D]
    m_hi = ahi + bhi + c[:, D:] + be_ref[:, D:]
    mu = sum_ref[0:1, :] * (1.0 / E)
    var = ssq_ref[0:1, :] * (1.0 / E) - mu * mu
    rstd = lax.rsqrt(var + 1e-5)
    scale = g_ref[...] * rstd
    shift = b_ref[...] - mu * scale
    hf = m_lo * scale[:, :D] + shift[:, :D]
    hs = m_hi * scale[:, D:] + shift[:, D:]
    out_ref[...] = jax.nn.sigmoid(hf) * jax.nn.softplus(hs)


def _gate(sa, sb, ef, w_aug, be, sums, ssqs, bg, bb):
    return _pc(
        _gate_body,
        grid=(E // BE,),
        in_specs=[
            pl.BlockSpec((BE, D), lambda i: (i, 0)),
            pl.BlockSpec((BE, D), lambda i: (i, 0)),
            pl.BlockSpec((BE, DEF), lambda i: (i, 0)),
            pl.BlockSpec((DEF, D2), lambda i: (0, 0)),
            pl.BlockSpec((1, D2), lambda i: (0, 0)),
            pl.BlockSpec((8, D2), lambda i: (0, 0)),
            pl.BlockSpec((8, D2), lambda i: (0, 0)),
            pl.BlockSpec((1, D2), lambda i: (0, 0)),
            pl.BlockSpec((1, D2), lambda i: (0, 0)),
        ],
        out_specs=pl.BlockSpec((BE, D), lambda i: (i, 0)),
        out_shape=jax.ShapeDtypeStruct((E, D), jnp.float32),
    )(sa, sb, ef, w_aug, be, sums, ssqs, bg, bb)


def _node_core(x_ref, hp_ref, g_ref, b_ref):
    h = hp_ref[0] + hp_ref[1]
    mu = jnp.mean(h, axis=0, keepdims=True)
    d = h - mu
    var = jnp.mean(d * d, axis=0, keepdims=True)
    hn = d * lax.rsqrt(var + 1e-5) * g_ref[...] + b_ref[...]
    return jax.nn.softplus(x_ref[...] + hn)


def _node_body(x_ref, hp_ref, g_ref, b_ref, ws_ref, bs_ref, wd_ref, bd_ref,
               x2_ref, hs_ref, hd_ref):
    x2 = _node_core(x_ref, hp_ref, g_ref, b_ref)
    x2_ref[...] = x2
    hs_ref[...] = _pack_words(
        jnp.dot(x2, ws_ref[...], preferred_element_type=jnp.float32) + bs_ref[...])
    hd_ref[...] = _pack_words(
        jnp.dot(x2, wd_ref[...], preferred_element_type=jnp.float32) + bd_ref[...])


def _node(x, hp, bg, bb, ws, bs, wd, bd):
    return _pc(
        _node_body,
        in_specs=[
            pl.BlockSpec((N, D), lambda: (0, 0)),
            pl.BlockSpec((2, N, D), lambda: (0, 0, 0)),
            pl.BlockSpec((1, D), lambda: (0, 0)),
            pl.BlockSpec((1, D), lambda: (0, 0)),
            pl.BlockSpec((D, D2), lambda: (0, 0)),
            pl.BlockSpec((1, D2), lambda: (0, 0)),
            pl.BlockSpec((D, D2), lambda: (0, 0)),
            pl.BlockSpec((1, D2), lambda: (0, 0)),
        ],
        out_specs=[
            pl.BlockSpec((N, D), lambda: (0, 0)),
            pl.BlockSpec((N, D), lambda: (0, 0)),
            pl.BlockSpec((N, D), lambda: (0, 0)),
        ],
        out_shape=[
            jax.ShapeDtypeStruct((N, D), jnp.float32),
            jax.ShapeDtypeStruct((N, D), jnp.int32),
            jax.ShapeDtypeStruct((N, D), jnp.int32),
        ],
    )(x, hp, bg, bb, ws, bs, wd, bd)


def _final_body(x_ref, hp_ref, g_ref, b_ref, fw_ref, fb_ref, out_ref):
    x2 = _node_core(x_ref, hp_ref, g_ref, b_ref)
    feat = jnp.mean(x2, axis=0, keepdims=True)
    out_ref[...] = jnp.dot(feat, fw_ref[...], preferred_element_type=jnp.float32) + fb_ref[...]


def _final(x, hp, bg, bb, fw, fb):
    return _pc(
        _final_body,
        in_specs=[
            pl.BlockSpec((N, D), lambda: (0, 0)),
            pl.BlockSpec((2, N, D), lambda: (0, 0, 0)),
            pl.BlockSpec((1, D), lambda: (0, 0)),
            pl.BlockSpec((1, D), lambda: (0, 0)),
            pl.BlockSpec((D, 1), lambda: (0, 0)),
            pl.BlockSpec((1, 1), lambda: (0, 0)),
        ],
        out_specs=pl.BlockSpec((1, 1), lambda: (0, 0)),
        out_shape=jax.ShapeDtypeStruct((1, 1), jnp.float32),
    )(x, hp, bg, bb, fw, fb)


def _b2i(a):
    return lax.bitcast_convert_type(a.reshape(a.shape[0], -1, 2), jnp.int32)


def _i2b(a):
    return lax.bitcast_convert_type(a, jnp.bfloat16).reshape(a.shape[0], -1)


def kernel(atom_types, edge_index, r, af_table, emb_W, emb_b, W_src, b_src,
           W_dst, b_dst, W_edge, b_edge, bn_m_g, bn_m_b, bn_g, bn_b,
           fc_out_W, fc_out_b):
    src = edge_index[0].astype(jnp.int32)
    dst = edge_index[1].astype(jnp.int32)

    ef = _rbf(r)
    af_pad = jnp.pad(af_table, ((0, 128 - af_table.shape[0]), (0, 0)))
    x, hs, hd = _embed(
        atom_types.astype(jnp.int32).reshape(N, 1), af_pad, emb_W,
        emb_b.reshape(1, D), W_src[0], b_src[0].reshape(1, D2),
        W_dst[0], b_dst[0].reshape(1, D2))

    out = None
    for i in range(LAYERS):
        sa, sb = _sc_gather(hs, hd, src, dst)
        w_aug = jnp.pad(W_edge[i], ((1, DEF - 1 - W_edge.shape[1]), (0, 0)))
        be = b_edge[i].reshape(1, D2)
        sums, ssqs = _stats(sa, sb, ef, w_aug, be)
        g = _gate(sa, sb, ef, w_aug, be, sums, ssqs,
                  bn_m_g[i].reshape(1, D2), bn_m_b[i].reshape(1, D2))
        hp = _sc_scatter(g, dst)
        bg = bn_g[i].reshape(1, D)
        bb = bn_b[i].reshape(1, D)
        if i < LAYERS - 1:
            x, hs, hd = _node(x, hp, bg, bb, W_src[i + 1],
                              b_src[i + 1].reshape(1, D2), W_dst[i + 1],
                              b_dst[i + 1].reshape(1, D2))
        else:
            out = _final(x, hp, bg, bb, fc_out_W, fc_out_b.reshape(1, 1))
    return out


# speedup vs baseline: 4.1542x; 1.2135x over previous
import functools

import jax
import jax.numpy as jnp
from jax import lax
from jax.experimental import pallas as pl
from jax.experimental.pallas import tpu as pltpu
from jax.experimental.pallas import tpu_sc as plsc

_pc = pl.pallas_call

N = 10000
E = 320000
D = 128
D2 = 256
DEF = 48
LAYERS = 3
NW = 32
EPW = E // NW
K = 80
NCH = EPW // K
RPT = N // 16

BE = 4000
BN_ = 2000


def _bondlen_body(r_ref, bl_ref):
    r = r_ref[...]
    bl_ref[...] = jnp.sqrt(jnp.sum(r * r, axis=1, keepdims=True))


def _bondlen(r):
    return _pc(
        _bondlen_body,
        grid=(E // BE,),
        in_specs=[pl.BlockSpec((BE, 3), lambda i: (i, 0))],
        out_specs=pl.BlockSpec((BE, 1), lambda i: (i, 0)),
        out_shape=jax.ShapeDtypeStruct((E, 1), jnp.float32),
    )(r)


def _rbf_expand(bl):
    lane_i = lax.broadcasted_iota(jnp.int32, (BE, 40), 1)
    c = lane_i.astype(jnp.float32) * (8.0 / 39.0)
    gamma = (39.0 / 8.0) ** 2
    return jnp.exp(-gamma * (bl - c) ** 2)


def _pack_words(h):
    u_lo = lax.bitcast_convert_type(h[:, :D], jnp.int32)
    u_hi = lax.bitcast_convert_type(h[:, D:], jnp.int32)
    r_lo = u_lo + 0x7FFF + jnp.bitwise_and(jnp.right_shift(u_lo, 16), 1)
    r_lo = jnp.bitwise_and(jnp.right_shift(r_lo, 16), 0xFFFF)
    r_hi = u_hi + 0x7FFF + jnp.bitwise_and(jnp.right_shift(u_hi, 16), 1)
    r_hi = jnp.bitwise_and(r_hi, -65536)
    return jnp.bitwise_or(r_lo, r_hi)


def _unpack_words(w):
    lo = lax.bitcast_convert_type(jnp.left_shift(w, 16), jnp.float32)
    hi = lax.bitcast_convert_type(jnp.bitwise_and(w, -65536), jnp.float32)
    return lo, hi


def _embed_body(at_ref, af_ref, ew_ref, eb_ref, ws_ref, bs_ref, wd_ref, bd_ref,
                x_ref, hs_ref, hd_ref):
    t = at_ref[...]
    oh = (lax.broadcasted_iota(jnp.int32, (BN_, 128), 1) == t).astype(jnp.float32)
    v = jnp.dot(oh, af_ref[...], preferred_element_type=jnp.float32)
    x = jnp.dot(v, ew_ref[...], preferred_element_type=jnp.float32) + eb_ref[...]
    x_ref[...] = x
    hs_ref[...] = _pack_words(
        jnp.dot(x, ws_ref[...], preferred_element_type=jnp.float32) + bs_ref[...])
    hd_ref[...] = _pack_words(
        jnp.dot(x, wd_ref[...], preferred_element_type=jnp.float32) + bd_ref[...])


def _embed(at, af_pad, ew, eb, ws, bs, wd, bd):
    da = af_pad.shape[1]
    return _pc(
        _embed_body,
        grid=(N // BN_,),
        in_specs=[
            pl.BlockSpec((BN_, 1), lambda i: (i, 0)),
            pl.BlockSpec((128, da), lambda i: (0, 0)),
            pl.BlockSpec((da, D), lambda i: (0, 0)),
            pl.BlockSpec((1, D), lambda i: (0, 0)),
            pl.BlockSpec((D, D2), lambda i: (0, 0)),
            pl.BlockSpec((1, D2), lambda i: (0, 0)),
            pl.BlockSpec((D, D2), lambda i: (0, 0)),
            pl.BlockSpec((1, D2), lambda i: (0, 0)),
        ],
        out_specs=[
            pl.BlockSpec((BN_, D), lambda i: (i, 0)),
            pl.BlockSpec((BN_, D), lambda i: (i, 0)),
            pl.BlockSpec((BN_, D), lambda i: (i, 0)),
        ],
        out_shape=[
            jax.ShapeDtypeStruct((N, D), jnp.float32),
            jax.ShapeDtypeStruct((N, D), jnp.int32),
            jax.ShapeDtypeStruct((N, D), jnp.int32),
        ],
    )(at, af_pad, ew, eb, ws, bs, wd, bd)


def _sc_gather(hs_i, hd_i, src, dst):
    DW = D2 // 2
    mesh = plsc.VectorSubcoreMesh(core_axis_name="c", subcore_axis_name="s")

    @functools.partial(
        pl.kernel,
        out_type=[
            jax.ShapeDtypeStruct((E, DW), jnp.int32),
            jax.ShapeDtypeStruct((E, DW), jnp.int32),
        ],
        mesh=mesh,
        scratch_types=[
            pltpu.VMEM((K,), jnp.int32),
            pltpu.VMEM((K,), jnp.int32),
            pltpu.VMEM((K,), jnp.int32),
            pltpu.VMEM((K,), jnp.int32),
            pltpu.VMEM((K, DW), jnp.int32),
            pltpu.VMEM((K, DW), jnp.int32),
            pltpu.VMEM((K, DW), jnp.int32),
            pltpu.VMEM((K, DW), jnp.int32),
            pltpu.SemaphoreType.DMA,
            pltpu.SemaphoreType.DMA,
            pltpu.SemaphoreType.DMA,
            pltpu.SemaphoreType.DMA,
        ],
    )
    def k(hs_hbm, hd_hbm, src_hbm, dst_hbm, ga_hbm, gb_hbm, isrc0, isrc1,
          idst0, idst1, a0, b0, a1, b1, sa0, sb0, sa1, sb1):
        cid = lax.axis_index("c")
        sid = lax.axis_index("s")
        base = (cid * 16 + sid) * EPW
        abufs, bbufs = (a0, a1), (b0, b1)
        isrcs, idsts = (isrc0, isrc1), (idst0, idst1)
        sas, sbs = (sa0, sa1), (sb0, sb1)

        def issue(j, p):
            e0 = base + j * K
            pltpu.sync_copy(src_hbm.at[pl.ds(e0, K)], isrcs[p])
            pltpu.sync_copy(dst_hbm.at[pl.ds(e0, K)], idsts[p])
            pltpu.async_copy(hs_hbm.at[isrcs[p]], abufs[p], sas[p])
            pltpu.async_copy(hd_hbm.at[idsts[p]], bbufs[p], sbs[p])

        def drain(j, p):
            e0 = base + j * K
            pltpu.make_async_copy(hs_hbm.at[isrcs[p]], abufs[p],
                                  sas[p]).wait()
            pltpu.sync_copy(abufs[p], ga_hbm.at[pl.ds(e0, K)])
            pltpu.make_async_copy(hd_hbm.at[idsts[p]], bbufs[p],
                                  sbs[p]).wait()
            pltpu.sync_copy(bbufs[p], gb_hbm.at[pl.ds(e0, K)])

        issue(0, 0)

        def pair(t, carry):
            issue(2 * t + 1, 1)
            drain(2 * t, 0)
            issue(2 * t + 2, 0)
            drain(2 * t + 1, 1)
            return carry

        lax.fori_loop(0, (NCH - 1) // 2, pair, 0)
        drain(NCH - 1, 0)

    return k(hs_i, hd_i, src, dst)


def _sc_scatter(g, dst):
    mesh = plsc.VectorSubcoreMesh(core_axis_name="c", subcore_axis_name="s")

    @functools.partial(
        pl.kernel,
        out_type=jax.ShapeDtypeStruct((2, 16, RPT, D), jnp.float32),
        mesh=mesh,
        scratch_types=[
            pltpu.VMEM((K,), jnp.int32),
            pltpu.VMEM((K,), jnp.int32),
            pltpu.VMEM((K, D), jnp.float32),
            pltpu.VMEM((K, D), jnp.float32),
            pltpu.VMEM((RPT // 5, D), jnp.float32),
            pltpu.VMEM_SHARED((N, D), jnp.float32),
            pltpu.SemaphoreType.DMA,
            pltpu.SemaphoreType.DMA,
        ],
    )
    def k(g_hbm, dst_hbm, out_hbm, idx0, idx1, g0, g1, zbuf, acc, sl0, sl1):
        cid = lax.axis_index("c")
        sid = lax.axis_index("s")
        wid = cid * 16 + sid
        idxs, gbufs, sls = (idx0, idx1), (g0, g1), (sl0, sl1)

        zr = RPT // 5

        def zrow(i, c2):
            for q in range(D // 16):
                zbuf[i, pl.ds(q * 16, 16)] = jnp.zeros((16,), jnp.float32)
            return c2

        lax.fori_loop(0, zr, zrow, 0)

        def zcp(i, c2):
            pltpu.sync_copy(zbuf, acc.at[pl.ds(sid * RPT + i * zr, zr)])
            return c2

        lax.fori_loop(0, 5, zcp, 0)
        plsc.subcore_barrier()

        def issue_load(j, p):
            e0 = wid * EPW + j * K
            pltpu.sync_copy(dst_hbm.at[pl.ds(e0, K)], idxs[p])
            pltpu.async_copy(g_hbm.at[pl.ds(e0, K)], gbufs[p], sls[p])

        def add(j, p):
            e0 = wid * EPW + j * K
            pltpu.make_async_copy(g_hbm.at[pl.ds(e0, K)], gbufs[p],
                                  sls[p]).wait()
            pltpu.sync_copy(gbufs[p], acc.at[idxs[p]], add=True)

        issue_load(0, 0)

        def pair(t, carry):
            issue_load(2 * t + 1, 1)
            add(2 * t, 0)
            issue_load(2 * t + 2, 0)
            add(2 * t + 1, 1)
            return carry

        lax.fori_loop(0, (NCH - 1) // 2, pair, 0)
        add(NCH - 1, 0)
        plsc.subcore_barrier()
        pltpu.sync_copy(acc.at[pl.ds(sid * RPT, RPT)], out_hbm.at[cid, sid])

    return k(g, dst).reshape(2, N, D)


def _stats_body(sa_ref, sb_ref, bl_ref, w_ref, be_ref, sum_ref, ssq_ref):
    i = pl.program_id(0)
    c = jnp.dot(_rbf_expand(bl_ref[...]), w_ref[...],
                preferred_element_type=jnp.float32)
    alo, ahi = _unpack_words(sa_ref[...])
    blo, bhi = _unpack_words(sb_ref[...])
    m_lo = alo + blo + c[:, :D] + be_ref[:, :D]
    m_hi = ahi + bhi + c[:, D:] + be_ref[:, D:]
    m = jnp.concatenate([m_lo, m_hi], axis=1)
    s0 = jnp.pad(jnp.sum(m, axis=0, keepdims=True), ((0, 7), (0, 0)))
    s1 = jnp.pad(jnp.sum(m * m, axis=0, keepdims=True), ((0, 7), (0, 0)))

    @pl.when(i == 0)
    def _():
        sum_ref[...] = jnp.zeros_like(sum_ref)
        ssq_ref[...] = jnp.zeros_like(ssq_ref)

    sum_ref[...] += s0
    ssq_ref[...] += s1


def _stats(sa, sb, bl, w, be):
    return _pc(
        _stats_body,
        grid=(E // BE,),
        in_specs=[
            pl.BlockSpec((BE, D), lambda i: (i, 0)),
            pl.BlockSpec((BE, D), lambda i: (i, 0)),
            pl.BlockSpec((BE, 1), lambda i: (i, 0)),
            pl.BlockSpec((40, D2), lambda i: (0, 0)),
            pl.BlockSpec((1, D2), lambda i: (0, 0)),
        ],
        out_specs=[
            pl.BlockSpec((8, D2), lambda i: (0, 0)),
            pl.BlockSpec((8, D2), lambda i: (0, 0)),
        ],
        out_shape=[
            jax.ShapeDtypeStruct((8, D2), jnp.float32),
            jax.ShapeDtypeStruct((8, D2), jnp.float32),
        ],
    )(sa, sb, bl, w, be)


def _gate_body(sa_ref, sb_ref, bl_ref, w_ref, be_ref, sum_ref, ssq_ref,
               g_ref, b_ref, out_ref):
    c = jnp.dot(_rbf_expand(bl_ref[...]), w_ref[...],
                preferred_element_type=jnp.float32)
    alo, ahi = _unpack_words(sa_ref[...])
    blo, bhi = _unpack_words(sb_ref[...])
    m_lo = alo + blo + c[:, :D] + be_ref[:, :D]
    m_hi = ahi + bhi + c[:, D:] + be_ref[:, D:]
    mu = sum_ref[0:1, :] * (1.0 / E)
    var = ssq_ref[0:1, :] * (1.0 / E) - mu * mu
    rstd = lax.rsqrt(var + 1e-5)
    scale = g_ref[...] * rstd
    shift = b_ref[...] - mu * scale
    hf = m_lo * scale[:, :D] + shift[:, :D]
    hs = m_hi * scale[:, D:] + shift[:, D:]
    out_ref[...] = jax.nn.sigmoid(hf) * jax.nn.softplus(hs)


def _gate(sa, sb, bl, w, be, sums, ssqs, bg, bb):
    return _pc(
        _gate_body,
        grid=(E // BE,),
        in_specs=[
            pl.BlockSpec((BE, D), lambda i: (i, 0)),
            pl.BlockSpec((BE, D), lambda i: (i, 0)),
            pl.BlockSpec((BE, 1), lambda i: (i, 0)),
            pl.BlockSpec((40, D2), lambda i: (0, 0)),
            pl.BlockSpec((1, D2), lambda i: (0, 0)),
            pl.BlockSpec((8, D2), lambda i: (0, 0)),
            pl.BlockSpec((8, D2), lambda i: (0, 0)),
            pl.BlockSpec((1, D2), lambda i: (0, 0)),
            pl.BlockSpec((1, D2), lambda i: (0, 0)),
        ],
        out_specs=pl.BlockSpec((BE, D), lambda i: (i, 0)),
        out_shape=jax.ShapeDtypeStruct((E, D), jnp.float32),
    )(sa, sb, bl, w, be, sums, ssqs, bg, bb)


def _node_core(x_ref, hp_ref, g_ref, b_ref):
    h = hp_ref[0] + hp_ref[1]
    mu = jnp.mean(h, axis=0, keepdims=True)
    d = h - mu
    var = jnp.mean(d * d, axis=0, keepdims=True)
    hn = d * lax.rsqrt(var + 1e-5) * g_ref[...] + b_ref[...]
    return jax.nn.softplus(x_ref[...] + hn)


def _node_body(x_ref, hp_ref, g_ref, b_ref, ws_ref, bs_ref, wd_ref, bd_ref,
               x2_ref, hs_ref, hd_ref):
    x2 = _node_core(x_ref, hp_ref, g_ref, b_ref)
    x2_ref[...] = x2
    hs_ref[...] = _pack_words(
        jnp.dot(x2, ws_ref[...], preferred_element_type=jnp.float32) + bs_ref[...])
    hd_ref[...] = _pack_words(
        jnp.dot(x2, wd_ref[...], preferred_element_type=jnp.float32) + bd_ref[...])


def _node(x, hp, bg, bb, ws, bs, wd, bd):
    return _pc(
        _node_body,
        in_specs=[
            pl.BlockSpec((N, D), lambda: (0, 0)),
            pl.BlockSpec((2, N, D), lambda: (0, 0, 0)),
            pl.BlockSpec((1, D), lambda: (0, 0)),
            pl.BlockSpec((1, D), lambda: (0, 0)),
            pl.BlockSpec((D, D2), lambda: (0, 0)),
            pl.BlockSpec((1, D2), lambda: (0, 0)),
            pl.BlockSpec((D, D2), lambda: (0, 0)),
            pl.BlockSpec((1, D2), lambda: (0, 0)),
        ],
        out_specs=[
            pl.BlockSpec((N, D), lambda: (0, 0)),
            pl.BlockSpec((N, D), lambda: (0, 0)),
            pl.BlockSpec((N, D), lambda: (0, 0)),
        ],
        out_shape=[
            jax.ShapeDtypeStruct((N, D), jnp.float32),
            jax.ShapeDtypeStruct((N, D), jnp.int32),
            jax.ShapeDtypeStruct((N, D), jnp.int32),
        ],
    )(x, hp, bg, bb, ws, bs, wd, bd)


def _final_body(x_ref, hp_ref, g_ref, b_ref, fw_ref, fb_ref, out_ref):
    x2 = _node_core(x_ref, hp_ref, g_ref, b_ref)
    feat = jnp.mean(x2, axis=0, keepdims=True)
    out_ref[...] = jnp.dot(feat, fw_ref[...], preferred_element_type=jnp.float32) + fb_ref[...]


def _final(x, hp, bg, bb, fw, fb):
    return _pc(
        _final_body,
        in_specs=[
            pl.BlockSpec((N, D), lambda: (0, 0)),
            pl.BlockSpec((2, N, D), lambda: (0, 0, 0)),
            pl.BlockSpec((1, D), lambda: (0, 0)),
            pl.BlockSpec((1, D), lambda: (0, 0)),
            pl.BlockSpec((D, 1), lambda: (0, 0)),
            pl.BlockSpec((1, 1), lambda: (0, 0)),
        ],
        out_specs=pl.BlockSpec((1, 1), lambda: (0, 0)),
        out_shape=jax.ShapeDtypeStruct((1, 1), jnp.float32),
    )(x, hp, bg, bb, fw, fb)


def _b2i(a):
    return lax.bitcast_convert_type(a.reshape(a.shape[0], -1, 2), jnp.int32)


def _i2b(a):
    return lax.bitcast_convert_type(a, jnp.bfloat16).reshape(a.shape[0], -1)


def kernel(atom_types, edge_index, r, af_table, emb_W, emb_b, W_src, b_src,
           W_dst, b_dst, W_edge, b_edge, bn_m_g, bn_m_b, bn_g, bn_b,
           fc_out_W, fc_out_b):
    src = edge_index[0].astype(jnp.int32)
    dst = edge_index[1].astype(jnp.int32)

    bl = _bondlen(r)
    af_pad = jnp.pad(af_table, ((0, 128 - af_table.shape[0]), (0, 0)))
    x, hs, hd = _embed(
        atom_types.astype(jnp.int32).reshape(N, 1), af_pad, emb_W,
        emb_b.reshape(1, D), W_src[0], b_src[0].reshape(1, D2),
        W_dst[0], b_dst[0].reshape(1, D2))

    out = None
    for i in range(LAYERS):
        sa, sb = _sc_gather(hs, hd, src, dst)
        be = b_edge[i].reshape(1, D2)
        sums, ssqs = _stats(sa, sb, bl, W_edge[i], be)
        g = _gate(sa, sb, bl, W_edge[i], be, sums, ssqs,
                  bn_m_g[i].reshape(1, D2), bn_m_b[i].reshape(1, D2))
        hp = _sc_scatter(g, dst)
        bg = bn_g[i].reshape(1, D)
        bb = bn_b[i].reshape(1, D)
        if i < LAYERS - 1:
            x, hs, hd = _node(x, hp, bg, bb, W_src[i + 1],
                              b_src[i + 1].reshape(1, D2), W_dst[i + 1],
                              b_dst[i + 1].reshape(1, D2))
        else:
            out = _final(x, hp, bg, bb, fc_out_W, fc_out_b.reshape(1, 1))
    return out


# speedup vs baseline: 4.1951x; 1.0099x over previous
import functools

import jax
import jax.numpy as jnp
from jax import lax
from jax.experimental import pallas as pl
from jax.experimental.pallas import tpu as pltpu
from jax.experimental.pallas import tpu_sc as plsc

_pc = pl.pallas_call

N = 10000
E = 320000
D = 128
D2 = 256
DEF = 48
LAYERS = 3
NW = 32
EPW = E // NW
K = 80
NCH = EPW // K
RPT = N // 16

BE = 4000
BN_ = 2000


def _bondlen_body(r_ref, bl_ref):
    r = r_ref[...]
    bl_ref[...] = jnp.sqrt(jnp.sum(r * r, axis=1, keepdims=True))


def _bondlen(r):
    return _pc(
        _bondlen_body,
        grid=(E // BE,),
        in_specs=[pl.BlockSpec((BE, 3), lambda i: (i, 0))],
        out_specs=pl.BlockSpec((BE, 1), lambda i: (i, 0)),
        out_shape=jax.ShapeDtypeStruct((E, 1), jnp.float32),
    )(r)


def _rbf_expand(bl):
    lane_i = lax.broadcasted_iota(jnp.int32, (BE, 40), 1)
    c = lane_i.astype(jnp.float32) * (8.0 / 39.0)
    gamma = (39.0 / 8.0) ** 2
    return jnp.exp(-gamma * (bl - c) ** 2)


def _pack_words(h):
    u_lo = lax.bitcast_convert_type(h[:, :D], jnp.int32)
    u_hi = lax.bitcast_convert_type(h[:, D:], jnp.int32)
    r_lo = u_lo + 0x7FFF + jnp.bitwise_and(jnp.right_shift(u_lo, 16), 1)
    r_lo = jnp.bitwise_and(jnp.right_shift(r_lo, 16), 0xFFFF)
    r_hi = u_hi + 0x7FFF + jnp.bitwise_and(jnp.right_shift(u_hi, 16), 1)
    r_hi = jnp.bitwise_and(r_hi, -65536)
    return jnp.bitwise_or(r_lo, r_hi)


def _unpack_words(w):
    lo = lax.bitcast_convert_type(jnp.left_shift(w, 16), jnp.float32)
    hi = lax.bitcast_convert_type(jnp.bitwise_and(w, -65536), jnp.float32)
    return lo, hi


def _embed_body(at_ref, af_ref, ew_ref, eb_ref, ws_ref, bs_ref, wd_ref, bd_ref,
                x_ref, hs_ref, hd_ref):
    t = at_ref[...]
    oh = (lax.broadcasted_iota(jnp.int32, (BN_, 128), 1) == t).astype(jnp.float32)
    v = jnp.dot(oh, af_ref[...], preferred_element_type=jnp.float32)
    x = jnp.dot(v, ew_ref[...], preferred_element_type=jnp.float32) + eb_ref[...]
    x_ref[...] = x
    hs_ref[...] = _pack_words(
        jnp.dot(x, ws_ref[...], preferred_element_type=jnp.float32) + bs_ref[...])
    hd_ref[...] = _pack_words(
        jnp.dot(x, wd_ref[...], preferred_element_type=jnp.float32) + bd_ref[...])


def _embed(at, af_pad, ew, eb, ws, bs, wd, bd):
    da = af_pad.shape[1]
    return _pc(
        _embed_body,
        grid=(N // BN_,),
        in_specs=[
            pl.BlockSpec((BN_, 1), lambda i: (i, 0)),
            pl.BlockSpec((128, da), lambda i: (0, 0)),
            pl.BlockSpec((da, D), lambda i: (0, 0)),
            pl.BlockSpec((1, D), lambda i: (0, 0)),
            pl.BlockSpec((D, D2), lambda i: (0, 0)),
            pl.BlockSpec((1, D2), lambda i: (0, 0)),
            pl.BlockSpec((D, D2), lambda i: (0, 0)),
            pl.BlockSpec((1, D2), lambda i: (0, 0)),
        ],
        out_specs=[
            pl.BlockSpec((BN_, D), lambda i: (i, 0)),
            pl.BlockSpec((BN_, D), lambda i: (i, 0)),
            pl.BlockSpec((BN_, D), lambda i: (i, 0)),
        ],
        out_shape=[
            jax.ShapeDtypeStruct((N, D), jnp.float32),
            jax.ShapeDtypeStruct((N, D), jnp.int32),
            jax.ShapeDtypeStruct((N, D), jnp.int32),
        ],
    )(at, af_pad, ew, eb, ws, bs, wd, bd)


def _sc_gather(hs_i, hd_i, src, dst):
    DW = D2 // 2
    mesh = plsc.VectorSubcoreMesh(core_axis_name="c", subcore_axis_name="s")

    @functools.partial(
        pl.kernel,
        out_type=[
            jax.ShapeDtypeStruct((E, DW), jnp.int32),
            jax.ShapeDtypeStruct((E, DW), jnp.int32),
        ],
        mesh=mesh,
        scratch_types=[
            pltpu.VMEM((K,), jnp.int32),
            pltpu.VMEM((K,), jnp.int32),
            pltpu.VMEM((K,), jnp.int32),
            pltpu.VMEM((K,), jnp.int32),
            pltpu.VMEM((K, DW), jnp.int32),
            pltpu.VMEM((K, DW), jnp.int32),
            pltpu.VMEM((K, DW), jnp.int32),
            pltpu.VMEM((K, DW), jnp.int32),
            pltpu.SemaphoreType.DMA,
            pltpu.SemaphoreType.DMA,
            pltpu.SemaphoreType.DMA,
            pltpu.SemaphoreType.DMA,
        ],
    )
    def k(hs_hbm, hd_hbm, src_hbm, dst_hbm, ga_hbm, gb_hbm, isrc0, isrc1,
          idst0, idst1, a0, b0, a1, b1, sa0, sb0, sa1, sb1):
        cid = lax.axis_index("c")
        sid = lax.axis_index("s")
        base = (cid * 16 + sid) * EPW
        abufs, bbufs = (a0, a1), (b0, b1)
        isrcs, idsts = (isrc0, isrc1), (idst0, idst1)
        sas, sbs = (sa0, sa1), (sb0, sb1)

        def issue(j, p):
            e0 = base + j * K
            pltpu.sync_copy(src_hbm.at[pl.ds(e0, K)], isrcs[p])
            pltpu.sync_copy(dst_hbm.at[pl.ds(e0, K)], idsts[p])
            pltpu.async_copy(hs_hbm.at[isrcs[p]], abufs[p], sas[p])
            pltpu.async_copy(hd_hbm.at[idsts[p]], bbufs[p], sbs[p])

        def drain(j, p):
            e0 = base + j * K
            pltpu.make_async_copy(hs_hbm.at[isrcs[p]], abufs[p],
                                  sas[p]).wait()
            pltpu.sync_copy(abufs[p], ga_hbm.at[pl.ds(e0, K)])
            pltpu.make_async_copy(hd_hbm.at[idsts[p]], bbufs[p],
                                  sbs[p]).wait()
            pltpu.sync_copy(bbufs[p], gb_hbm.at[pl.ds(e0, K)])

        issue(0, 0)

        def pair(t, carry):
            issue(2 * t + 1, 1)
            drain(2 * t, 0)
            issue(2 * t + 2, 0)
            drain(2 * t + 1, 1)
            return carry

        lax.fori_loop(0, (NCH - 1) // 2, pair, 0)
        drain(NCH - 1, 0)

    return k(hs_i, hd_i, src, dst)


def _sc_scatter(g, dst):
    mesh = plsc.VectorSubcoreMesh(core_axis_name="c", subcore_axis_name="s")

    @functools.partial(
        pl.kernel,
        out_type=jax.ShapeDtypeStruct((2, N, D), jnp.float32),
        mesh=mesh,
        scratch_types=[
            pltpu.VMEM((K,), jnp.int32),
            pltpu.VMEM((K,), jnp.int32),
            pltpu.VMEM((K, D), jnp.float32),
            pltpu.VMEM((K, D), jnp.float32),
            pltpu.VMEM((RPT // 5, D), jnp.float32),
            pltpu.VMEM_SHARED((N, D), jnp.float32),
            pltpu.SemaphoreType.DMA,
            pltpu.SemaphoreType.DMA,
        ],
    )
    def k(g_hbm, dst_hbm, out_hbm, idx0, idx1, g0, g1, zbuf, acc, sl0, sl1):
        cid = lax.axis_index("c")
        sid = lax.axis_index("s")
        wid = cid * 16 + sid
        idxs, gbufs, sls = (idx0, idx1), (g0, g1), (sl0, sl1)

        zr = RPT // 5

        def zrow(i, c2):
            for q in range(D // 16):
                zbuf[i, pl.ds(q * 16, 16)] = jnp.zeros((16,), jnp.float32)
            return c2

        lax.fori_loop(0, zr, zrow, 0)

        def zcp(i, c2):
            pltpu.sync_copy(zbuf, acc.at[pl.ds(sid * RPT + i * zr, zr)])
            return c2

        lax.fori_loop(0, 5, zcp, 0)
        plsc.subcore_barrier()

        def issue_load(j, p):
            e0 = wid * EPW + j * K
            pltpu.sync_copy(dst_hbm.at[pl.ds(e0, K)], idxs[p])
            pltpu.async_copy(g_hbm.at[pl.ds(e0, K)], gbufs[p], sls[p])

        def add(j, p):
            e0 = wid * EPW + j * K
            pltpu.make_async_copy(g_hbm.at[pl.ds(e0, K)], gbufs[p],
                                  sls[p]).wait()
            pltpu.sync_copy(gbufs[p], acc.at[idxs[p]], add=True)

        issue_load(0, 0)

        def pair(t, carry):
            issue_load(2 * t + 1, 1)
            add(2 * t, 0)
            issue_load(2 * t + 2, 0)
            add(2 * t + 1, 1)
            return carry

        lax.fori_loop(0, (NCH - 1) // 2, pair, 0)
        add(NCH - 1, 0)
        plsc.subcore_barrier()

        @pl.when(sid < 15)
        def _():
            pltpu.sync_copy(acc.at[pl.ds(sid * 624, 624)],
                            out_hbm.at[cid, pl.ds(sid * 624, 624)])

        @pl.when(sid == 15)
        def _():
            pltpu.sync_copy(acc.at[pl.ds(15 * 624, 640)],
                            out_hbm.at[cid, pl.ds(15 * 624, 640)])

    return k(g, dst)


def _stats_body(sa_ref, sb_ref, bl_ref, w_ref, be_ref, sum_ref, ssq_ref):
    i = pl.program_id(0)
    c = jnp.dot(_rbf_expand(bl_ref[...]), w_ref[...],
                preferred_element_type=jnp.float32)
    alo, ahi = _unpack_words(sa_ref[...])
    blo, bhi = _unpack_words(sb_ref[...])
    m_lo = alo + blo + c[:, :D] + be_ref[:, :D]
    m_hi = ahi + bhi + c[:, D:] + be_ref[:, D:]
    m = jnp.concatenate([m_lo, m_hi], axis=1)
    s0 = jnp.pad(jnp.sum(m, axis=0, keepdims=True), ((0, 7), (0, 0)))
    s1 = jnp.pad(jnp.sum(m * m, axis=0, keepdims=True), ((0, 7), (0, 0)))

    @pl.when(i == 0)
    def _():
        sum_ref[...] = jnp.zeros_like(sum_ref)
        ssq_ref[...] = jnp.zeros_like(ssq_ref)

    sum_ref[...] += s0
    ssq_ref[...] += s1


def _stats(sa, sb, bl, w, be):
    return _pc(
        _stats_body,
        grid=(E // BE,),
        in_specs=[
            pl.BlockSpec((BE, D), lambda i: (i, 0)),
            pl.BlockSpec((BE, D), lambda i: (i, 0)),
            pl.BlockSpec((BE, 1), lambda i: (i, 0)),
            pl.BlockSpec((40, D2), lambda i: (0, 0)),
            pl.BlockSpec((1, D2), lambda i: (0, 0)),
        ],
        out_specs=[
            pl.BlockSpec((8, D2), lambda i: (0, 0)),
            pl.BlockSpec((8, D2), lambda i: (0, 0)),
        ],
        out_shape=[
            jax.ShapeDtypeStruct((8, D2), jnp.float32),
            jax.ShapeDtypeStruct((8, D2), jnp.float32),
        ],
    )(sa, sb, bl, w, be)


def _gate_body(sa_ref, sb_ref, bl_ref, w_ref, be_ref, sum_ref, ssq_ref,
               g_ref, b_ref, out_ref):
    c = jnp.dot(_rbf_expand(bl_ref[...]), w_ref[...],
                preferred_element_type=jnp.float32)
    alo, ahi = _unpack_words(sa_ref[...])
    blo, bhi = _unpack_words(sb_ref[...])
    m_lo = alo + blo + c[:, :D] + be_ref[:, :D]
    m_hi = ahi + bhi + c[:, D:] + be_ref[:, D:]
    mu = sum_ref[0:1, :] * (1.0 / E)
    var = ssq_ref[0:1, :] * (1.0 / E) - mu * mu
    rstd = lax.rsqrt(var + 1e-5)
    scale = g_ref[...] * rstd
    shift = b_ref[...] - mu * scale
    hf = m_lo * scale[:, :D] + shift[:, :D]
    hs = m_hi * scale[:, D:] + shift[:, D:]
    out_ref[...] = jax.nn.sigmoid(hf) * jax.nn.softplus(hs)


def _gate(sa, sb, bl, w, be, sums, ssqs, bg, bb):
    return _pc(
        _gate_body,
        grid=(E // BE,),
        in_specs=[
            pl.BlockSpec((BE, D), lambda i: (i, 0)),
            pl.BlockSpec((BE, D), lambda i: (i, 0)),
            pl.BlockSpec((BE, 1), lambda i: (i, 0)),
            pl.BlockSpec((40, D2), lambda i: (0, 0)),
            pl.BlockSpec((1, D2), lambda i: (0, 0)),
            pl.BlockSpec((8, D2), lambda i: (0, 0)),
            pl.BlockSpec((8, D2), lambda i: (0, 0)),
            pl.BlockSpec((1, D2), lambda i: (0, 0)),
            pl.BlockSpec((1, D2), lambda i: (0, 0)),
        ],
        out_specs=pl.BlockSpec((BE, D), lambda i: (i, 0)),
        out_shape=jax.ShapeDtypeStruct((E, D), jnp.float32),
    )(sa, sb, bl, w, be, sums, ssqs, bg, bb)


def _node_core(x_ref, hp_ref, g_ref, b_ref):
    h = hp_ref[0] + hp_ref[1]
    mu = jnp.mean(h, axis=0, keepdims=True)
    d = h - mu
    var = jnp.mean(d * d, axis=0, keepdims=True)
    hn = d * lax.rsqrt(var + 1e-5) * g_ref[...] + b_ref[...]
    return jax.nn.softplus(x_ref[...] + hn)


def _node_body(x_ref, hp_ref, g_ref, b_ref, ws_ref, bs_ref, wd_ref, bd_ref,
               x2_ref, hs_ref, hd_ref):
    x2 = _node_core(x_ref, hp_ref, g_ref, b_ref)
    x2_ref[...] = x2
    hs_ref[...] = _pack_words(
        jnp.dot(x2, ws_ref[...], preferred_element_type=jnp.float32) + bs_ref[...])
    hd_ref[...] = _pack_words(
        jnp.dot(x2, wd_ref[...], preferred_element_type=jnp.float32) + bd_ref[...])


def _node(x, hp, bg, bb, ws, bs, wd, bd):
    return _pc(
        _node_body,
        in_specs=[
            pl.BlockSpec((N, D), lambda: (0, 0)),
            pl.BlockSpec((2, N, D), lambda: (0, 0, 0)),
            pl.BlockSpec((1, D), lambda: (0, 0)),
            pl.BlockSpec((1, D), lambda: (0, 0)),
            pl.BlockSpec((D, D2), lambda: (0, 0)),
            pl.BlockSpec((1, D2), lambda: (0, 0)),
            pl.BlockSpec((D, D2), lambda: (0, 0)),
            pl.BlockSpec((1, D2), lambda: (0, 0)),
        ],
        out_specs=[
            pl.BlockSpec((N, D), lambda: (0, 0)),
            pl.BlockSpec((N, D), lambda: (0, 0)),
            pl.BlockSpec((N, D), lambda: (0, 0)),
        ],
        out_shape=[
            jax.ShapeDtypeStruct((N, D), jnp.float32),
            jax.ShapeDtypeStruct((N, D), jnp.int32),
            jax.ShapeDtypeStruct((N, D), jnp.int32),
        ],
    )(x, hp, bg, bb, ws, bs, wd, bd)


def _final_body(x_ref, hp_ref, g_ref, b_ref, fw_ref, fb_ref, out_ref):
    x2 = _node_core(x_ref, hp_ref, g_ref, b_ref)
    feat = jnp.mean(x2, axis=0, keepdims=True)
    out_ref[...] = jnp.dot(feat, fw_ref[...], preferred_element_type=jnp.float32) + fb_ref[...]


def _final(x, hp, bg, bb, fw, fb):
    return _pc(
        _final_body,
        in_specs=[
            pl.BlockSpec((N, D), lambda: (0, 0)),
            pl.BlockSpec((2, N, D), lambda: (0, 0, 0)),
            pl.BlockSpec((1, D), lambda: (0, 0)),
            pl.BlockSpec((1, D), lambda: (0, 0)),
            pl.BlockSpec((D, 1), lambda: (0, 0)),
            pl.BlockSpec((1, 1), lambda: (0, 0)),
        ],
        out_specs=pl.BlockSpec((1, 1), lambda: (0, 0)),
        out_shape=jax.ShapeDtypeStruct((1, 1), jnp.float32),
    )(x, hp, bg, bb, fw, fb)


def _b2i(a):
    return lax.bitcast_convert_type(a.reshape(a.shape[0], -1, 2), jnp.int32)


def _i2b(a):
    return lax.bitcast_convert_type(a, jnp.bfloat16).reshape(a.shape[0], -1)


def kernel(atom_types, edge_index, r, af_table, emb_W, emb_b, W_src, b_src,
           W_dst, b_dst, W_edge, b_edge, bn_m_g, bn_m_b, bn_g, bn_b,
           fc_out_W, fc_out_b):
    src = edge_index[0].astype(jnp.int32)
    dst = edge_index[1].astype(jnp.int32)

    bl = _bondlen(r)
    af_pad = jnp.pad(af_table, ((0, 128 - af_table.shape[0]), (0, 0)))
    x, hs, hd = _embed(
        atom_types.astype(jnp.int32).reshape(N, 1), af_pad, emb_W,
        emb_b.reshape(1, D), W_src[0], b_src[0].reshape(1, D2),
        W_dst[0], b_dst[0].reshape(1, D2))

    out = None
    for i in range(LAYERS):
        sa, sb = _sc_gather(hs, hd, src, dst)
        be = b_edge[i].reshape(1, D2)
        sums, ssqs = _stats(sa, sb, bl, W_edge[i], be)
        g = _gate(sa, sb, bl, W_edge[i], be, sums, ssqs,
                  bn_m_g[i].reshape(1, D2), bn_m_b[i].reshape(1, D2))
        hp = _sc_scatter(g, dst)
        bg = bn_g[i].reshape(1, D)
        bb = bn_b[i].reshape(1, D)
        if i < LAYERS - 1:
            x, hs, hd = _node(x, hp, bg, bb, W_src[i + 1],
                              b_src[i + 1].reshape(1, D2), W_dst[i + 1],
                              b_dst[i + 1].reshape(1, D2))
        else:
            out = _final(x, hp, bg, bb, fc_out_W, fc_out_b.reshape(1, 1))
    return out


# speedup vs baseline: 4.4678x; 1.0650x over previous
import functools

import jax
import jax.numpy as jnp
from jax import lax
from jax.experimental import pallas as pl
from jax.experimental.pallas import tpu as pltpu
from jax.experimental.pallas import tpu_sc as plsc

_pc = pl.pallas_call

N = 10000
E = 320000
D = 128
D2 = 256
DEF = 48
LAYERS = 3
NW = 32
EPW = E // NW
K = 80
NCH = EPW // K
RPT = N // 16

BE = 4000
BN_ = 2000


def _bondlen_body(r_ref, bl_ref):
    r = r_ref[...]
    bl_ref[...] = jnp.sqrt(jnp.sum(r * r, axis=1, keepdims=True))


def _bondlen(r):
    return _pc(
        _bondlen_body,
        grid=(E // BE,),
        in_specs=[pl.BlockSpec((BE, 3), lambda i: (i, 0))],
        out_specs=pl.BlockSpec((BE, 1), lambda i: (i, 0)),
        out_shape=jax.ShapeDtypeStruct((E, 1), jnp.float32),
    )(r)


def _rbf_expand(bl):
    lane_i = lax.broadcasted_iota(jnp.int32, (BE, 40), 1)
    c = lane_i.astype(jnp.float32) * (8.0 / 39.0)
    gamma = (39.0 / 8.0) ** 2
    return jnp.exp(-gamma * (bl - c) ** 2)


def _pack_words(h):
    u_lo = lax.bitcast_convert_type(h[:, :D], jnp.int32)
    u_hi = lax.bitcast_convert_type(h[:, D:], jnp.int32)
    r_lo = u_lo + 0x7FFF + jnp.bitwise_and(jnp.right_shift(u_lo, 16), 1)
    r_lo = jnp.bitwise_and(jnp.right_shift(r_lo, 16), 0xFFFF)
    r_hi = u_hi + 0x7FFF + jnp.bitwise_and(jnp.right_shift(u_hi, 16), 1)
    r_hi = jnp.bitwise_and(r_hi, -65536)
    return jnp.bitwise_or(r_lo, r_hi)


def _unpack_words(w):
    lo = lax.bitcast_convert_type(jnp.left_shift(w, 16), jnp.float32)
    hi = lax.bitcast_convert_type(jnp.bitwise_and(w, -65536), jnp.float32)
    return lo, hi


def _embed_body(at_ref, af_ref, ew_ref, eb_ref, ws_ref, bs_ref, wd_ref, bd_ref,
                x_ref, hs_ref, hd_ref):
    t = at_ref[...]
    oh = (lax.broadcasted_iota(jnp.int32, (BN_, 128), 1) == t).astype(jnp.float32)
    v = jnp.dot(oh, af_ref[...], preferred_element_type=jnp.float32)
    x = jnp.dot(v, ew_ref[...], preferred_element_type=jnp.float32) + eb_ref[...]
    x_ref[...] = x
    hs_ref[...] = _pack_words(
        jnp.dot(x, ws_ref[...], preferred_element_type=jnp.float32) + bs_ref[...])
    hd_ref[...] = _pack_words(
        jnp.dot(x, wd_ref[...], preferred_element_type=jnp.float32) + bd_ref[...])


def _embed(at, af_pad, ew, eb, ws, bs, wd, bd):
    da = af_pad.shape[1]
    return _pc(
        _embed_body,
        grid=(N // BN_,),
        in_specs=[
            pl.BlockSpec((BN_, 1), lambda i: (i, 0)),
            pl.BlockSpec((128, da), lambda i: (0, 0)),
            pl.BlockSpec((da, D), lambda i: (0, 0)),
            pl.BlockSpec((1, D), lambda i: (0, 0)),
            pl.BlockSpec((D, D2), lambda i: (0, 0)),
            pl.BlockSpec((1, D2), lambda i: (0, 0)),
            pl.BlockSpec((D, D2), lambda i: (0, 0)),
            pl.BlockSpec((1, D2), lambda i: (0, 0)),
        ],
        out_specs=[
            pl.BlockSpec((BN_, D), lambda i: (i, 0)),
            pl.BlockSpec((BN_, D), lambda i: (i, 0)),
            pl.BlockSpec((BN_, D), lambda i: (i, 0)),
        ],
        out_shape=[
            jax.ShapeDtypeStruct((N, D), jnp.float32),
            jax.ShapeDtypeStruct((N, D), jnp.int32),
            jax.ShapeDtypeStruct((N, D), jnp.int32),
        ],
    )(at, af_pad, ew, eb, ws, bs, wd, bd)


def _sc_gather(hs_i, hd_i, src, dst):
    DW = D2 // 2
    mesh = plsc.VectorSubcoreMesh(core_axis_name="c", subcore_axis_name="s")

    @functools.partial(
        pl.kernel,
        out_type=[
            jax.ShapeDtypeStruct((E, DW), jnp.int32),
            jax.ShapeDtypeStruct((E, DW), jnp.int32),
        ],
        mesh=mesh,
        scratch_types=[
            pltpu.VMEM((NCH, K), jnp.int32),
            pltpu.VMEM((NCH, K), jnp.int32),
            pltpu.VMEM((K, DW), jnp.int32),
            pltpu.VMEM((K, DW), jnp.int32),
            pltpu.VMEM((K, DW), jnp.int32),
            pltpu.VMEM((K, DW), jnp.int32),
            pltpu.SemaphoreType.DMA,
            pltpu.SemaphoreType.DMA,
            pltpu.SemaphoreType.DMA,
            pltpu.SemaphoreType.DMA,
        ],
    )
    def k(hs_hbm, hd_hbm, src_hbm, dst_hbm, ga_hbm, gb_hbm, isrc, idst,
          a0, b0, a1, b1, sa0, sb0, sa1, sb1):
        cid = lax.axis_index("c")
        sid = lax.axis_index("s")
        wid = cid * 16 + sid
        base = wid * EPW
        abufs, bbufs = (a0, a1), (b0, b1)
        sas, sbs = (sa0, sa1), (sb0, sb1)

        pltpu.sync_copy(src_hbm.at[wid], isrc)
        pltpu.sync_copy(dst_hbm.at[wid], idst)

        def issue(j, p):
            pltpu.async_copy(hs_hbm.at[isrc.at[j]], abufs[p], sas[p])
            pltpu.async_copy(hd_hbm.at[idst.at[j]], bbufs[p], sbs[p])

        def drain(j, p):
            e0 = base + j * K
            pltpu.make_async_copy(hs_hbm.at[isrc.at[j]], abufs[p],
                                  sas[p]).wait()
            pltpu.sync_copy(abufs[p], ga_hbm.at[pl.ds(e0, K)])
            pltpu.make_async_copy(hd_hbm.at[idst.at[j]], bbufs[p],
                                  sbs[p]).wait()
            pltpu.sync_copy(bbufs[p], gb_hbm.at[pl.ds(e0, K)])

        issue(0, 0)

        def pair(t, carry):
            issue(2 * t + 1, 1)
            drain(2 * t, 0)
            issue(2 * t + 2, 0)
            drain(2 * t + 1, 1)
            return carry

        lax.fori_loop(0, (NCH - 1) // 2, pair, 0)
        drain(NCH - 1, 0)

    return k(hs_i, hd_i, src.reshape(NW, NCH, K), dst.reshape(NW, NCH, K))


def _sc_scatter(g, dst):
    mesh = plsc.VectorSubcoreMesh(core_axis_name="c", subcore_axis_name="s")

    @functools.partial(
        pl.kernel,
        out_type=jax.ShapeDtypeStruct((2, N, D), jnp.float32),
        mesh=mesh,
        scratch_types=[
            pltpu.VMEM((NCH, K), jnp.int32),
            pltpu.VMEM((K, D), jnp.float32),
            pltpu.VMEM((K, D), jnp.float32),
            pltpu.VMEM((RPT // 25, D), jnp.float32),
            pltpu.VMEM_SHARED((N, D), jnp.float32),
            pltpu.SemaphoreType.DMA,
            pltpu.SemaphoreType.DMA,
        ],
    )
    def k(g_hbm, dst_hbm, out_hbm, idx, g0, g1, zbuf, acc, sl0, sl1):
        cid = lax.axis_index("c")
        sid = lax.axis_index("s")
        wid = cid * 16 + sid
        gbufs, sls = (g0, g1), (sl0, sl1)
        pltpu.sync_copy(dst_hbm.at[wid], idx)

        zr = RPT // 25

        def zrow(i, c2):
            for q in range(D // 16):
                zbuf[i, pl.ds(q * 16, 16)] = jnp.zeros((16,), jnp.float32)
            return c2

        lax.fori_loop(0, zr, zrow, 0)

        def zcp(i, c2):
            pltpu.sync_copy(zbuf, acc.at[pl.ds(sid * RPT + i * zr, zr)])
            return c2

        lax.fori_loop(0, 25, zcp, 0)
        plsc.subcore_barrier()

        def issue_load(j, p):
            e0 = wid * EPW + j * K
            pltpu.async_copy(g_hbm.at[pl.ds(e0, K)], gbufs[p], sls[p])

        def add(j, p):
            e0 = wid * EPW + j * K
            pltpu.make_async_copy(g_hbm.at[pl.ds(e0, K)], gbufs[p],
                                  sls[p]).wait()
            pltpu.sync_copy(gbufs[p], acc.at[idx.at[j]], add=True)

        issue_load(0, 0)

        def pair(t, carry):
            issue_load(2 * t + 1, 1)
            add(2 * t, 0)
            issue_load(2 * t + 2, 0)
            add(2 * t + 1, 1)
            return carry

        lax.fori_loop(0, (NCH - 1) // 2, pair, 0)
        add(NCH - 1, 0)
        plsc.subcore_barrier()

        @pl.when(sid < 15)
        def _():
            pltpu.sync_copy(acc.at[pl.ds(sid * 624, 624)],
                            out_hbm.at[cid, pl.ds(sid * 624, 624)])

        @pl.when(sid == 15)
        def _():
            pltpu.sync_copy(acc.at[pl.ds(15 * 624, 640)],
                            out_hbm.at[cid, pl.ds(15 * 624, 640)])

    return k(g, dst.reshape(NW, NCH, K))


def _stats_body(sa_ref, sb_ref, bl_ref, w_ref, be_ref, sum_ref, ssq_ref):
    i = pl.program_id(0)
    c = jnp.dot(_rbf_expand(bl_ref[...]), w_ref[...],
                preferred_element_type=jnp.float32)
    alo, ahi = _unpack_words(sa_ref[...])
    blo, bhi = _unpack_words(sb_ref[...])
    m_lo = alo + blo + c[:, :D] + be_ref[:, :D]
    m_hi = ahi + bhi + c[:, D:] + be_ref[:, D:]
    m = jnp.concatenate([m_lo, m_hi], axis=1)
    s0 = jnp.pad(jnp.sum(m, axis=0, keepdims=True), ((0, 7), (0, 0)))
    s1 = jnp.pad(jnp.sum(m * m, axis=0, keepdims=True), ((0, 7), (0, 0)))

    @pl.when(i == 0)
    def _():
        sum_ref[...] = jnp.zeros_like(sum_ref)
        ssq_ref[...] = jnp.zeros_like(ssq_ref)

    sum_ref[...] += s0
    ssq_ref[...] += s1


def _stats(sa, sb, bl, w, be):
    return _pc(
        _stats_body,
        grid=(E // BE,),
        in_specs=[
            pl.BlockSpec((BE, D), lambda i: (i, 0)),
            pl.BlockSpec((BE, D), lambda i: (i, 0)),
            pl.BlockSpec((BE, 1), lambda i: (i, 0)),
            pl.BlockSpec((40, D2), lambda i: (0, 0)),
            pl.BlockSpec((1, D2), lambda i: (0, 0)),
        ],
        out_specs=[
            pl.BlockSpec((8, D2), lambda i: (0, 0)),
            pl.BlockSpec((8, D2), lambda i: (0, 0)),
        ],
        out_shape=[
            jax.ShapeDtypeStruct((8, D2), jnp.float32),
            jax.ShapeDtypeStruct((8, D2), jnp.float32),
        ],
    )(sa, sb, bl, w, be)


def _gate_body(sa_ref, sb_ref, bl_ref, w_ref, be_ref, sum_ref, ssq_ref,
               g_ref, b_ref, out_ref):
    c = jnp.dot(_rbf_expand(bl_ref[...]), w_ref[...],
                preferred_element_type=jnp.float32)
    alo, ahi = _unpack_words(sa_ref[...])
    blo, bhi = _unpack_words(sb_ref[...])
    m_lo = alo + blo + c[:, :D] + be_ref[:, :D]
    m_hi = ahi + bhi + c[:, D:] + be_ref[:, D:]
    mu = sum_ref[0:1, :] * (1.0 / E)
    var = ssq_ref[0:1, :] * (1.0 / E) - mu * mu
    rstd = lax.rsqrt(var + 1e-5)
    scale = g_ref[...] * rstd
    shift = b_ref[...] - mu * scale
    hf = m_lo * scale[:, :D] + shift[:, :D]
    hs = m_hi * scale[:, D:] + shift[:, D:]
    out_ref[...] = jax.nn.sigmoid(hf) * jax.nn.softplus(hs)


def _gate(sa, sb, bl, w, be, sums, ssqs, bg, bb):
    return _pc(
        _gate_body,
        grid=(E // BE,),
        in_specs=[
            pl.BlockSpec((BE, D), lambda i: (i, 0)),
            pl.BlockSpec((BE, D), lambda i: (i, 0)),
            pl.BlockSpec((BE, 1), lambda i: (i, 0)),
            pl.BlockSpec((40, D2), lambda i: (0, 0)),
            pl.BlockSpec((1, D2), lambda i: (0, 0)),
            pl.BlockSpec((8, D2), lambda i: (0, 0)),
            pl.BlockSpec((8, D2), lambda i: (0, 0)),
            pl.BlockSpec((1, D2), lambda i: (0, 0)),
            pl.BlockSpec((1, D2), lambda i: (0, 0)),
        ],
        out_specs=pl.BlockSpec((BE, D), lambda i: (i, 0)),
        out_shape=jax.ShapeDtypeStruct((E, D), jnp.float32),
    )(sa, sb, bl, w, be, sums, ssqs, bg, bb)


def _node_core(x_ref, hp_ref, g_ref, b_ref):
    h = hp_ref[0] + hp_ref[1]
    mu = jnp.mean(h, axis=0, keepdims=True)
    d = h - mu
    var = jnp.mean(d * d, axis=0, keepdims=True)
    hn = d * lax.rsqrt(var + 1e-5) * g_ref[...] + b_ref[...]
    return jax.nn.softplus(x_ref[...] + hn)


def _node_body(x_ref, hp_ref, g_ref, b_ref, ws_ref, bs_ref, wd_ref, bd_ref,
               x2_ref, hs_ref, hd_ref):
    x2 = _node_core(x_ref, hp_ref, g_ref, b_ref)
    x2_ref[...] = x2
    hs_ref[...] = _pack_words(
        jnp.dot(x2, ws_ref[...], preferred_element_type=jnp.float32) + bs_ref[...])
    hd_ref[...] = _pack_words(
        jnp.dot(x2, wd_ref[...], preferred_element_type=jnp.float32) + bd_ref[...])


def _node(x, hp, bg, bb, ws, bs, wd, bd):
    return _pc(
        _node_body,
        in_specs=[
            pl.BlockSpec((N, D), lambda: (0, 0)),
            pl.BlockSpec((2, N, D), lambda: (0, 0, 0)),
            pl.BlockSpec((1, D), lambda: (0, 0)),
            pl.BlockSpec((1, D), lambda: (0, 0)),
            pl.BlockSpec((D, D2), lambda: (0, 0)),
            pl.BlockSpec((1, D2), lambda: (0, 0)),
            pl.BlockSpec((D, D2), lambda: (0, 0)),
            pl.BlockSpec((1, D2), lambda: (0, 0)),
        ],
        out_specs=[
            pl.BlockSpec((N, D), lambda: (0, 0)),
            pl.BlockSpec((N, D), lambda: (0, 0)),
            pl.BlockSpec((N, D), lambda: (0, 0)),
        ],
        out_shape=[
            jax.ShapeDtypeStruct((N, D), jnp.float32),
            jax.ShapeDtypeStruct((N, D), jnp.int32),
            jax.ShapeDtypeStruct((N, D), jnp.int32),
        ],
    )(x, hp, bg, bb, ws, bs, wd, bd)


def _final_body(x_ref, hp_ref, g_ref, b_ref, fw_ref, fb_ref, out_ref):
    x2 = _node_core(x_ref, hp_ref, g_ref, b_ref)
    feat = jnp.mean(x2, axis=0, keepdims=True)
    out_ref[...] = jnp.dot(feat, fw_ref[...], preferred_element_type=jnp.float32) + fb_ref[...]


def _final(x, hp, bg, bb, fw, fb):
    return _pc(
        _final_body,
        in_specs=[
            pl.BlockSpec((N, D), lambda: (0, 0)),
            pl.BlockSpec((2, N, D), lambda: (0, 0, 0)),
            pl.BlockSpec((1, D), lambda: (0, 0)),
            pl.BlockSpec((1, D), lambda: (0, 0)),
            pl.BlockSpec((D, 1), lambda: (0, 0)),
            pl.BlockSpec((1, 1), lambda: (0, 0)),
        ],
        out_specs=pl.BlockSpec((1, 1), lambda: (0, 0)),
        out_shape=jax.ShapeDtypeStruct((1, 1), jnp.float32),
    )(x, hp, bg, bb, fw, fb)


def _b2i(a):
    return lax.bitcast_convert_type(a.reshape(a.shape[0], -1, 2), jnp.int32)


def _i2b(a):
    return lax.bitcast_convert_type(a, jnp.bfloat16).reshape(a.shape[0], -1)


def kernel(atom_types, edge_index, r, af_table, emb_W, emb_b, W_src, b_src,
           W_dst, b_dst, W_edge, b_edge, bn_m_g, bn_m_b, bn_g, bn_b,
           fc_out_W, fc_out_b):
    src = edge_index[0].astype(jnp.int32)
    dst = edge_index[1].astype(jnp.int32)

    bl = _bondlen(r)
    af_pad = jnp.pad(af_table, ((0, 128 - af_table.shape[0]), (0, 0)))
    x, hs, hd = _embed(
        atom_types.astype(jnp.int32).reshape(N, 1), af_pad, emb_W,
        emb_b.reshape(1, D), W_src[0], b_src[0].reshape(1, D2),
        W_dst[0], b_dst[0].reshape(1, D2))

    out = None
    for i in range(LAYERS):
        sa, sb = _sc_gather(hs, hd, src, dst)
        be = b_edge[i].reshape(1, D2)
        sums, ssqs = _stats(sa, sb, bl, W_edge[i], be)
        g = _gate(sa, sb, bl, W_edge[i], be, sums, ssqs,
                  bn_m_g[i].reshape(1, D2), bn_m_b[i].reshape(1, D2))
        hp = _sc_scatter(g, dst)
        bg = bn_g[i].reshape(1, D)
        bb = bn_b[i].reshape(1, D)
        if i < LAYERS - 1:
            x, hs, hd = _node(x, hp, bg, bb, W_src[i + 1],
                              b_src[i + 1].reshape(1, D2), W_dst[i + 1],
                              b_dst[i + 1].reshape(1, D2))
        else:
            out = _final(x, hp, bg, bb, fc_out_W, fc_out_b.reshape(1, 1))
    return out


# speedup vs baseline: 4.4685x; 1.0001x over previous
import functools

import jax
import jax.numpy as jnp
from jax import lax
from jax.experimental import pallas as pl
from jax.experimental.pallas import tpu as pltpu
from jax.experimental.pallas import tpu_sc as plsc

_pc = pl.pallas_call

N = 10000
E = 320000
D = 128
D2 = 256
LAYERS = 3
NW = 32
EPW = E // NW
K = 80
NCH = EPW // K
RPT = N // 16

BE = 4000
BN_ = 2000


def _bondlen_body(r_ref, bl_ref):
    r = r_ref[...]
    bl_ref[...] = jnp.sqrt(jnp.sum(r * r, axis=1, keepdims=True))


def _bondlen(r):
    return _pc(
        _bondlen_body,
        grid=(E // BE,),
        in_specs=[pl.BlockSpec((BE, 3), lambda i: (i, 0))],
        out_specs=pl.BlockSpec((BE, 1), lambda i: (i, 0)),
        out_shape=jax.ShapeDtypeStruct((E, 1), jnp.float32),
    )(r)


def _rbf_expand(bl):
    lane_i = lax.broadcasted_iota(jnp.int32, (BE, 40), 1)
    c = lane_i.astype(jnp.float32) * (8.0 / 39.0)
    gamma = (39.0 / 8.0) ** 2
    return jnp.exp(-gamma * (bl - c) ** 2)


def _pack_words(h):
    u_lo = lax.bitcast_convert_type(h[:, :D], jnp.int32)
    u_hi = lax.bitcast_convert_type(h[:, D:], jnp.int32)
    r_lo = u_lo + 0x7FFF + jnp.bitwise_and(jnp.right_shift(u_lo, 16), 1)
    r_lo = jnp.bitwise_and(jnp.right_shift(r_lo, 16), 0xFFFF)
    r_hi = u_hi + 0x7FFF + jnp.bitwise_and(jnp.right_shift(u_hi, 16), 1)
    r_hi = jnp.bitwise_and(r_hi, -65536)
    return jnp.bitwise_or(r_lo, r_hi)


def _unpack_words(w):
    lo = lax.bitcast_convert_type(jnp.left_shift(w, 16), jnp.float32)
    hi = lax.bitcast_convert_type(jnp.bitwise_and(w, -65536), jnp.float32)
    return lo, hi


def _embed_body(at_ref, af_ref, ew_ref, eb_ref, ws_ref, bs_ref, wd_ref, bd_ref,
                x_ref, hs_ref, hd_ref):
    t = at_ref[...]
    oh = (lax.broadcasted_iota(jnp.int32, (BN_, 128), 1) == t).astype(jnp.float32)
    v = jnp.dot(oh, af_ref[...], preferred_element_type=jnp.float32)
    x = jnp.dot(v, ew_ref[...], preferred_element_type=jnp.float32) + eb_ref[...]
    x_ref[...] = x
    hs_ref[...] = _pack_words(
        jnp.dot(x, ws_ref[...], preferred_element_type=jnp.float32) + bs_ref[...])
    hd_ref[...] = _pack_words(
        jnp.dot(x, wd_ref[...], preferred_element_type=jnp.float32) + bd_ref[...])


def _embed(at, af_pad, ew, eb, ws, bs, wd, bd):
    da = af_pad.shape[1]
    return _pc(
        _embed_body,
        grid=(N // BN_,),
        in_specs=[
            pl.BlockSpec((BN_, 1), lambda i: (i, 0)),
            pl.BlockSpec((128, da), lambda i: (0, 0)),
            pl.BlockSpec((da, D), lambda i: (0, 0)),
            pl.BlockSpec((1, D), lambda i: (0, 0)),
            pl.BlockSpec((D, D2), lambda i: (0, 0)),
            pl.BlockSpec((1, D2), lambda i: (0, 0)),
            pl.BlockSpec((D, D2), lambda i: (0, 0)),
            pl.BlockSpec((1, D2), lambda i: (0, 0)),
        ],
        out_specs=[
            pl.BlockSpec((BN_, D), lambda i: (i, 0)),
            pl.BlockSpec((BN_, D), lambda i: (i, 0)),
            pl.BlockSpec((BN_, D), lambda i: (i, 0)),
        ],
        out_shape=[
            jax.ShapeDtypeStruct((N, D), jnp.float32),
            jax.ShapeDtypeStruct((N, D), jnp.int32),
            jax.ShapeDtypeStruct((N, D), jnp.int32),
        ],
    )(at, af_pad, ew, eb, ws, bs, wd, bd)


def _sc_gather(hs_i, hd_i, src, dst):
    DW = D2 // 2
    mesh = plsc.VectorSubcoreMesh(core_axis_name="c", subcore_axis_name="s")

    @functools.partial(
        pl.kernel,
        out_type=[
            jax.ShapeDtypeStruct((E, DW), jnp.int32),
            jax.ShapeDtypeStruct((E, DW), jnp.int32),
        ],
        mesh=mesh,
        scratch_types=[
            pltpu.VMEM((NCH, K), jnp.int32),
            pltpu.VMEM((NCH, K), jnp.int32),
            pltpu.VMEM((K, DW), jnp.int32),
            pltpu.VMEM((K, DW), jnp.int32),
            pltpu.VMEM((K, DW), jnp.int32),
            pltpu.VMEM((K, DW), jnp.int32),
            pltpu.SemaphoreType.DMA,
            pltpu.SemaphoreType.DMA,
            pltpu.SemaphoreType.DMA,
            pltpu.SemaphoreType.DMA,
        ],
    )
    def k(hs_hbm, hd_hbm, src_hbm, dst_hbm, ga_hbm, gb_hbm, isrc, idst,
          a0, b0, a1, b1, sa0, sb0, sa1, sb1):
        cid = lax.axis_index("c")
        sid = lax.axis_index("s")
        wid = cid * 16 + sid
        base = wid * EPW
        abufs, bbufs = (a0, a1), (b0, b1)
        sas, sbs = (sa0, sa1), (sb0, sb1)

        pltpu.sync_copy(src_hbm.at[wid], isrc)
        pltpu.sync_copy(dst_hbm.at[wid], idst)

        def issue(j, p):
            pltpu.async_copy(hs_hbm.at[isrc.at[j]], abufs[p], sas[p])
            pltpu.async_copy(hd_hbm.at[idst.at[j]], bbufs[p], sbs[p])

        def drain(j, p):
            e0 = base + j * K
            pltpu.make_async_copy(hs_hbm.at[isrc.at[j]], abufs[p],
                                  sas[p]).wait()
            pltpu.sync_copy(abufs[p], ga_hbm.at[pl.ds(e0, K)])
            pltpu.make_async_copy(hd_hbm.at[idst.at[j]], bbufs[p],
                                  sbs[p]).wait()
            pltpu.sync_copy(bbufs[p], gb_hbm.at[pl.ds(e0, K)])

        issue(0, 0)

        def pair(t, carry):
            issue(2 * t + 1, 1)
            drain(2 * t, 0)
            issue(2 * t + 2, 0)
            drain(2 * t + 1, 1)
            return carry

        lax.fori_loop(0, (NCH - 1) // 2, pair, 0)
        drain(NCH - 1, 0)

    return k(hs_i, hd_i, src.reshape(NW, NCH, K), dst.reshape(NW, NCH, K))


def _sc_scatter(g, dst):
    mesh = plsc.VectorSubcoreMesh(core_axis_name="c", subcore_axis_name="s")

    @functools.partial(
        pl.kernel,
        out_type=jax.ShapeDtypeStruct((2, N, D), jnp.float32),
        mesh=mesh,
        scratch_types=[
            pltpu.VMEM((NCH, K), jnp.int32),
            pltpu.VMEM((K, D), jnp.float32),
            pltpu.VMEM((K, D), jnp.float32),
            pltpu.VMEM((RPT // 25, D), jnp.float32),
            pltpu.VMEM_SHARED((N, D), jnp.float32),
            pltpu.SemaphoreType.DMA,
            pltpu.SemaphoreType.DMA,
        ],
    )
    def k(g_hbm, dst_hbm, out_hbm, idx, g0, g1, zbuf, acc, sl0, sl1):
        cid = lax.axis_index("c")
        sid = lax.axis_index("s")
        wid = cid * 16 + sid
        gbufs, sls = (g0, g1), (sl0, sl1)
        pltpu.sync_copy(dst_hbm.at[wid], idx)

        zr = RPT // 25

        def zrow(i, c2):
            for q in range(D // 16):
                zbuf[i, pl.ds(q * 16, 16)] = jnp.zeros((16,), jnp.float32)
            return c2

        lax.fori_loop(0, zr, zrow, 0)

        def zcp(i, c2):
            pltpu.sync_copy(zbuf, acc.at[pl.ds(sid * RPT + i * zr, zr)])
            return c2

        lax.fori_loop(0, 25, zcp, 0)
        plsc.subcore_barrier()

        def issue_load(j, p):
            e0 = wid * EPW + j * K
            pltpu.async_copy(g_hbm.at[pl.ds(e0, K)], gbufs[p], sls[p])

        def add(j, p):
            e0 = wid * EPW + j * K
            pltpu.make_async_copy(g_hbm.at[pl.ds(e0, K)], gbufs[p],
                                  sls[p]).wait()
            pltpu.sync_copy(gbufs[p], acc.at[idx.at[j]], add=True)

        issue_load(0, 0)

        def pair(t, carry):
            issue_load(2 * t + 1, 1)
            add(2 * t, 0)
            issue_load(2 * t + 2, 0)
            add(2 * t + 1, 1)
            return carry

        lax.fori_loop(0, (NCH - 1) // 2, pair, 0)
        add(NCH - 1, 0)
        plsc.subcore_barrier()

        @pl.when(sid < 15)
        def _():
            pltpu.sync_copy(acc.at[pl.ds(sid * 624, 624)],
                            out_hbm.at[cid, pl.ds(sid * 624, 624)])

        @pl.when(sid == 15)
        def _():
            pltpu.sync_copy(acc.at[pl.ds(15 * 624, 640)],
                            out_hbm.at[cid, pl.ds(15 * 624, 640)])

    return k(g, dst.reshape(NW, NCH, K))


def _stats_body(sa_ref, sb_ref, bl_ref, w_ref, be_ref, sum_ref, ssq_ref):
    i = pl.program_id(0)
    c = jnp.dot(_rbf_expand(bl_ref[...]), w_ref[...],
                preferred_element_type=jnp.float32)
    alo, ahi = _unpack_words(sa_ref[...])
    blo, bhi = _unpack_words(sb_ref[...])
    m_lo = alo + blo + c[:, :D] + be_ref[:, :D]
    m_hi = ahi + bhi + c[:, D:] + be_ref[:, D:]
    m = jnp.concatenate([m_lo, m_hi], axis=1)
    s0 = jnp.pad(jnp.sum(m, axis=0, keepdims=True), ((0, 7), (0, 0)))
    s1 = jnp.pad(jnp.sum(m * m, axis=0, keepdims=True), ((0, 7), (0, 0)))

    @pl.when(i == 0)
    def _():
        sum_ref[...] = jnp.zeros_like(sum_ref)
        ssq_ref[...] = jnp.zeros_like(ssq_ref)

    sum_ref[...] += s0
    ssq_ref[...] += s1


def _stats(sa, sb, bl, w, be):
    return _pc(
        _stats_body,
        grid=(E // BE,),
        in_specs=[
            pl.BlockSpec((BE, D), lambda i: (i, 0)),
            pl.BlockSpec((BE, D), lambda i: (i, 0)),
            pl.BlockSpec((BE, 1), lambda i: (i, 0)),
            pl.BlockSpec((40, D2), lambda i: (0, 0)),
            pl.BlockSpec((1, D2), lambda i: (0, 0)),
        ],
        out_specs=[
            pl.BlockSpec((8, D2), lambda i: (0, 0)),
            pl.BlockSpec((8, D2), lambda i: (0, 0)),
        ],
        out_shape=[
            jax.ShapeDtypeStruct((8, D2), jnp.float32),
            jax.ShapeDtypeStruct((8, D2), jnp.float32),
        ],
    )(sa, sb, bl, w, be)


def _gate_body(sa_ref, sb_ref, bl_ref, w_ref, be_ref, sum_ref, ssq_ref,
               g_ref, b_ref, out_ref):
    c = jnp.dot(_rbf_expand(bl_ref[...]), w_ref[...],
                preferred_element_type=jnp.float32)
    alo, ahi = _unpack_words(sa_ref[...])
    blo, bhi = _unpack_words(sb_ref[...])
    m_lo = alo + blo + c[:, :D] + be_ref[:, :D]
    m_hi = ahi + bhi + c[:, D:] + be_ref[:, D:]
    mu = sum_ref[0:1, :] * (1.0 / E)
    var = ssq_ref[0:1, :] * (1.0 / E) - mu * mu
    rstd = lax.rsqrt(var + 1e-5)
    scale = g_ref[...] * rstd
    shift = b_ref[...] - mu * scale
    hf = m_lo * scale[:, :D] + shift[:, :D]
    hs = m_hi * scale[:, D:] + shift[:, D:]
    out_ref[...] = jax.nn.sigmoid(hf) * jax.nn.softplus(hs)


def _gate(sa, sb, bl, w, be, sums, ssqs, bg, bb):
    return _pc(
        _gate_body,
        grid=(E // BE,),
        in_specs=[
            pl.BlockSpec((BE, D), lambda i: (i, 0)),
            pl.BlockSpec((BE, D), lambda i: (i, 0)),
            pl.BlockSpec((BE, 1), lambda i: (i, 0)),
            pl.BlockSpec((40, D2), lambda i: (0, 0)),
            pl.BlockSpec((1, D2), lambda i: (0, 0)),
            pl.BlockSpec((8, D2), lambda i: (0, 0)),
            pl.BlockSpec((8, D2), lambda i: (0, 0)),
            pl.BlockSpec((1, D2), lambda i: (0, 0)),
            pl.BlockSpec((1, D2), lambda i: (0, 0)),
        ],
        out_specs=pl.BlockSpec((BE, D), lambda i: (i, 0)),
        out_shape=jax.ShapeDtypeStruct((E, D), jnp.float32),
    )(sa, sb, bl, w, be, sums, ssqs, bg, bb)


def _node_core(x_ref, hp_ref, g_ref, b_ref):
    h = hp_ref[0] + hp_ref[1]
    mu = jnp.mean(h, axis=0, keepdims=True)
    d = h - mu
    var = jnp.mean(d * d, axis=0, keepdims=True)
    hn = d * lax.rsqrt(var + 1e-5) * g_ref[...] + b_ref[...]
    return jax.nn.softplus(x_ref[...] + hn)


def _node_body(x_ref, hp_ref, g_ref, b_ref, ws_ref, bs_ref, wd_ref, bd_ref,
               x2_ref, hs_ref, hd_ref):
    x2 = _node_core(x_ref, hp_ref, g_ref, b_ref)
    x2_ref[...] = x2
    hs_ref[...] = _pack_words(
        jnp.dot(x2, ws_ref[...], preferred_element_type=jnp.float32) + bs_ref[...])
    hd_ref[...] = _pack_words(
        jnp.dot(x2, wd_ref[...], preferred_element_type=jnp.float32) + bd_ref[...])


def _node(x, hp, bg, bb, ws, bs, wd, bd):
    return _pc(
        _node_body,
        in_specs=[
            pl.BlockSpec((N, D), lambda: (0, 0)),
            pl.BlockSpec((2, N, D), lambda: (0, 0, 0)),
            pl.BlockSpec((1, D), lambda: (0, 0)),
            pl.BlockSpec((1, D), lambda: (0, 0)),
            pl.BlockSpec((D, D2), lambda: (0, 0)),
            pl.BlockSpec((1, D2), lambda: (0, 0)),
            pl.BlockSpec((D, D2), lambda: (0, 0)),
            pl.BlockSpec((1, D2), lambda: (0, 0)),
        ],
        out_specs=[
            pl.BlockSpec((N, D), lambda: (0, 0)),
            pl.BlockSpec((N, D), lambda: (0, 0)),
            pl.BlockSpec((N, D), lambda: (0, 0)),
        ],
        out_shape=[
            jax.ShapeDtypeStruct((N, D), jnp.float32),
            jax.ShapeDtypeStruct((N, D), jnp.int32),
            jax.ShapeDtypeStruct((N, D), jnp.int32),
        ],
    )(x, hp, bg, bb, ws, bs, wd, bd)


def _final_body(x_ref, hp_ref, g_ref, b_ref, fw_ref, fb_ref, out_ref):
    x2 = _node_core(x_ref, hp_ref, g_ref, b_ref)
    feat = jnp.mean(x2, axis=0, keepdims=True)
    out_ref[...] = jnp.dot(feat, fw_ref[...], preferred_element_type=jnp.float32) + fb_ref[...]


def _final(x, hp, bg, bb, fw, fb):
    return _pc(
        _final_body,
        in_specs=[
            pl.BlockSpec((N, D), lambda: (0, 0)),
            pl.BlockSpec((2, N, D), lambda: (0, 0, 0)),
            pl.BlockSpec((1, D), lambda: (0, 0)),
            pl.BlockSpec((1, D), lambda: (0, 0)),
            pl.BlockSpec((D, 1), lambda: (0, 0)),
            pl.BlockSpec((1, 1), lambda: (0, 0)),
        ],
        out_specs=pl.BlockSpec((1, 1), lambda: (0, 0)),
        out_shape=jax.ShapeDtypeStruct((1, 1), jnp.float32),
    )(x, hp, bg, bb, fw, fb)


def kernel(atom_types, edge_index, r, af_table, emb_W, emb_b, W_src, b_src,
           W_dst, b_dst, W_edge, b_edge, bn_m_g, bn_m_b, bn_g, bn_b,
           fc_out_W, fc_out_b):
    src = edge_index[0].astype(jnp.int32)
    dst = edge_index[1].astype(jnp.int32)

    bl = _bondlen(r)
    af_pad = jnp.pad(af_table, ((0, 128 - af_table.shape[0]), (0, 0)))
    x, hs, hd = _embed(
        atom_types.astype(jnp.int32).reshape(N, 1), af_pad, emb_W,
        emb_b.reshape(1, D), W_src[0], b_src[0].reshape(1, D2),
        W_dst[0], b_dst[0].reshape(1, D2))

    out = None
    for i in range(LAYERS):
        sa, sb = _sc_gather(hs, hd, src, dst)
        be = b_edge[i].reshape(1, D2)
        sums, ssqs = _stats(sa, sb, bl, W_edge[i], be)
        g = _gate(sa, sb, bl, W_edge[i], be, sums, ssqs,
                  bn_m_g[i].reshape(1, D2), bn_m_b[i].reshape(1, D2))
        hp = _sc_scatter(g, dst)
        bg = bn_g[i].reshape(1, D)
        bb = bn_b[i].reshape(1, D)
        if i < LAYERS - 1:
            x, hs, hd = _node(x, hp, bg, bb, W_src[i + 1],
                              b_src[i + 1].reshape(1, D2), W_dst[i + 1],
                              b_dst[i + 1].reshape(1, D2))
        else:
            out = _final(x, hp, bg, bb, fc_out_W, fc_out_b.reshape(1, 1))
    return out
